# Initial kernel scaffold; baseline (speedup 1.0000x reference)
import jax
import jax.numpy as jnp
from jax.experimental import pallas as pl


def kernel(x, edge_index, W1, b1, W2, b2, Wm1, bm1, Wm2, bm2):
    raise NotImplementedError("write your pallas kernel here")


# speedup vs baseline: 6.2735x; 6.2735x over previous
import jax
import jax.numpy as jnp
from jax import lax
from jax.experimental import pallas as pl
from jax.experimental.pallas import tpu as pltpu
from jax.experimental.pallas import tpu_sc as plsc

N = 10000
E = 320000
NPAD = 10240
NC, NS, L = 2, 16, 16
NW = NC * NS
CH = 128
NCHUNK = 79
G0, G1 = 40, 39
EPT = NCHUNK * CH
EPAD = NW * EPT
RPT = NPAD // NS

_MESH = plsc.VectorSubcoreMesh(core_axis_name="c", subcore_axis_name="s",
                               num_cores=NC, num_subcores=NS)

_F32 = jnp.float32


def _worker(cid, sid):
    return cid * NS + sid


def _deg_body(dst_hbm, out_hbm, idx_v, ones_v, zb_v, acc_sh, sem):
    cid = lax.axis_index("c")
    sid = lax.axis_index("s")
    w = _worker(cid, sid)

    def fill_ones(i, c):
        ones_v[pl.ds(i * L, L)] = jnp.full((L,), 1.0, _F32)
        return c

    lax.fori_loop(0, CH // L, fill_ones, 0)

    def fill_zero(i, c):
        zb_v[pl.ds(i * L, L)] = jnp.zeros((L,), _F32)
        return c

    lax.fori_loop(0, RPT // L, fill_zero, 0)
    pltpu.sync_copy(zb_v, acc_sh.at[pl.ds(sid * RPT, RPT)])
    plsc.subcore_barrier()

    pltpu.async_copy(dst_hbm.at[w], idx_v, sem).wait()

    def body(j, c):
        pltpu.sync_copy(ones_v, acc_sh.at[idx_v.at[j]], add=True)
        return c

    lax.fori_loop(0, NCHUNK, body, 0)
    plsc.subcore_barrier()
    pltpu.sync_copy(acc_sh.at[pl.ds(sid * RPT, RPT)],
                    out_hbm.at[cid, pl.ds(sid * RPT, RPT)])


_deg_call = pl.kernel(
    _deg_body,
    out_type=jax.ShapeDtypeStruct((NC, NPAD), _F32),
    mesh=_MESH,
    scratch_types=[
        pltpu.VMEM((NCHUNK, CH), jnp.int32),
        pltpu.VMEM((CH,), _F32),
        pltpu.VMEM((RPT,), _F32),
        pltpu.VMEM_SHARED((NPAD,), _F32),
        pltpu.SemaphoreType.DMA,
    ],
)


def _agg_body(g_hbm, src_hbm, dst_hbm, out_hbm, sidx, didx, buf, acc_sh, sem):
    cid = lax.axis_index("c")
    sid = lax.axis_index("s")
    w = _worker(cid, sid)

    def fill_zero(i, c):
        r = i // 8
        col = i % 8
        buf[0, r, pl.ds(col * L, L)] = jnp.zeros((L,), _F32)
        return c

    lax.fori_loop(0, CH * 8, fill_zero, 0)
    for t in range(RPT // CH):
        pltpu.sync_copy(buf.at[0], acc_sh.at[pl.ds(sid * RPT + t * CH, CH)])
    plsc.subcore_barrier()

    for jbase, gcount in ((0, G0), (G0, G1)):
        pltpu.async_copy(src_hbm.at[w, pl.ds(jbase, gcount)],
                         sidx.at[pl.ds(0, gcount)], sem).wait()
        pltpu.async_copy(dst_hbm.at[w, pl.ds(jbase, gcount)],
                         didx.at[pl.ds(0, gcount)], sem).wait()

        pltpu.async_copy(g_hbm.at[sidx.at[0]], buf.at[0], sem)

        def body(j, c):
            pltpu.make_async_copy(g_hbm.at[sidx.at[j]], buf.at[j % 2],
                                  sem).wait()

            @pl.when(j + 1 < gcount)
            def _():
                pltpu.async_copy(g_hbm.at[sidx.at[j + 1]],
                                 buf.at[(j + 1) % 2], sem)

            pltpu.sync_copy(buf.at[j % 2], acc_sh.at[didx.at[j]], add=True)
            return c

        lax.fori_loop(0, gcount, body, 0)

    plsc.subcore_barrier()
    pltpu.sync_copy(acc_sh.at[pl.ds(sid * RPT, RPT)],
                    out_hbm.at[cid, pl.ds(sid * RPT, RPT)])


_agg_call = pl.kernel(
    _agg_body,
    out_type=jax.ShapeDtypeStruct((NC, NPAD, 128), _F32),
    mesh=_MESH,
    scratch_types=[
        pltpu.VMEM((G0, CH), jnp.int32),
        pltpu.VMEM((G0, CH), jnp.int32),
        pltpu.VMEM((2, CH, 128), _F32),
        pltpu.VMEM_SHARED((NPAD, 128), _F32),
        pltpu.SemaphoreType.DMA,
    ],
)


def _edge_body(a_hbm, b_hbm, src_hbm, dst_hbm, out_hbm, sidx, didx, buf, sem):
    cid = lax.axis_index("c")
    sid = lax.axis_index("s")
    w = _worker(cid, sid)
    base = w * EPT

    pltpu.async_copy(src_hbm.at[w], sidx, sem).wait()
    pltpu.async_copy(dst_hbm.at[w], didx, sem).wait()

    def body(j, c):
        pltpu.async_copy(a_hbm.at[sidx.at[j]], buf.at[j % 2], sem).wait()
        pltpu.async_copy(b_hbm.at[didx.at[j]], buf.at[j % 2], sem,
                         add=True).wait()
        pltpu.sync_copy(buf.at[j % 2], out_hbm.at[pl.ds(base + j * CH, CH)])
        return c

    lax.fori_loop(0, NCHUNK, body, 0)


_edge_call = pl.kernel(
    _edge_body,
    out_type=jax.ShapeDtypeStruct((EPAD, 128), _F32),
    mesh=_MESH,
    scratch_types=[
        pltpu.VMEM((NCHUNK, CH), jnp.int32),
        pltpu.VMEM((NCHUNK, CH), jnp.int32),
        pltpu.VMEM((2, CH, 128), _F32),
        pltpu.SemaphoreType.DMA,
    ],
)


_PREC = lax.Precision.HIGHEST


def _mm_body(x_ref, w_ref, o_ref):
    o_ref[:] = jnp.dot(x_ref[:], w_ref[:], preferred_element_type=_F32,
                       precision=_PREC)


def _tc_matmul(x, w, rows_per_block=1280):
    m = x.shape[0]
    grid = m // rows_per_block
    return pl.pallas_call(
        _mm_body,
        grid=(grid,),
        in_specs=[
            pl.BlockSpec((rows_per_block, x.shape[1]), lambda i: (i, 0)),
            pl.BlockSpec(w.shape, lambda i: (0, 0)),
        ],
        out_specs=pl.BlockSpec((rows_per_block, w.shape[1]), lambda i: (i, 0)),
        out_shape=jax.ShapeDtypeStruct((m, w.shape[1]), _F32),
    )(x, w)


def _scale_body(h_ref, d0_ref, d1_ref, g_ref, dinv_ref):
    dinv = lax.rsqrt(d0_ref[:] + d1_ref[:] + 1.0)
    dinv_ref[:] = dinv
    g_ref[:] = h_ref[:] * dinv


def _tc_scale(h, d0, d1):
    grid = NPAD // 1280
    return pl.pallas_call(
        _scale_body,
        grid=(grid,),
        in_specs=[
            pl.BlockSpec((1280, 128), lambda i: (i, 0)),
            pl.BlockSpec((1280, 1), lambda i: (i, 0)),
            pl.BlockSpec((1280, 1), lambda i: (i, 0)),
        ],
        out_specs=[
            pl.BlockSpec((1280, 128), lambda i: (i, 0)),
            pl.BlockSpec((1280, 1), lambda i: (i, 0)),
        ],
        out_shape=[
            jax.ShapeDtypeStruct((NPAD, 128), _F32),
            jax.ShapeDtypeStruct((NPAD, 1), _F32),
        ],
    )(h, d0, d1)


def _layer_body(a0_ref, a1_ref, g_ref, dinv_ref, b_ref, w_ref, o_ref):
    dinv = dinv_ref[:]
    h = (a0_ref[:] + a1_ref[:] + g_ref[:]) * dinv + b_ref[:]
    h = jnp.maximum(h, 0.0)
    o_ref[:] = jnp.dot(h, w_ref[:], preferred_element_type=_F32,
                       precision=_PREC) * dinv


def _tc_layer(agg, g, dinv, b, w):
    grid = NPAD // 1280
    return pl.pallas_call(
        _layer_body,
        grid=(grid,),
        in_specs=[
            pl.BlockSpec((1280, 128), lambda i: (i, 0)),
            pl.BlockSpec((1280, 128), lambda i: (i, 0)),
            pl.BlockSpec((1280, 128), lambda i: (i, 0)),
            pl.BlockSpec((1280, 1), lambda i: (i, 0)),
            pl.BlockSpec((1, 128), lambda i: (0, 0)),
            pl.BlockSpec((128, 128), lambda i: (0, 0)),
        ],
        out_specs=pl.BlockSpec((1280, 128), lambda i: (i, 0)),
        out_shape=jax.ShapeDtypeStruct((NPAD, 128), _F32),
    )(agg[0], agg[1], g, dinv, b, w)


def _final_node_body(a0_ref, a1_ref, g_ref, dinv_ref, b_ref, wa_ref, wb_ref,
                     oa_ref, ob_ref):
    dinv = dinv_ref[:]
    h = (a0_ref[:] + a1_ref[:] + g_ref[:]) * dinv + b_ref[:]
    h = jnp.maximum(h, 0.0)
    oa_ref[:] = jnp.dot(h, wa_ref[:], preferred_element_type=_F32,
                        precision=_PREC)
    ob_ref[:] = jnp.dot(h, wb_ref[:], preferred_element_type=_F32,
                        precision=_PREC)


def _tc_final_node(agg, g, dinv, b, wa, wb):
    grid = NPAD // 1280
    return pl.pallas_call(
        _final_node_body,
        grid=(grid,),
        in_specs=[
            pl.BlockSpec((1280, 128), lambda i: (i, 0)),
            pl.BlockSpec((1280, 128), lambda i: (i, 0)),
            pl.BlockSpec((1280, 128), lambda i: (i, 0)),
            pl.BlockSpec((1280, 1), lambda i: (i, 0)),
            pl.BlockSpec((1, 128), lambda i: (0, 0)),
            pl.BlockSpec((128, 128), lambda i: (0, 0)),
            pl.BlockSpec((128, 128), lambda i: (0, 0)),
        ],
        out_specs=[
            pl.BlockSpec((1280, 128), lambda i: (i, 0)),
            pl.BlockSpec((1280, 128), lambda i: (i, 0)),
        ],
        out_shape=[
            jax.ShapeDtypeStruct((NPAD, 128), _F32),
            jax.ShapeDtypeStruct((NPAD, 128), _F32),
        ],
    )(agg[0], agg[1], g, dinv, b, wa, wb)


def _edge_mlp_body(s_ref, b1_ref, w2_ref, b2_ref, o_ref):
    z = jnp.maximum(s_ref[:] + b1_ref[:], 0.0)
    o_ref[:] = jnp.dot(z, w2_ref[:], preferred_element_type=_F32,
                       precision=_PREC) + b2_ref[:]


def _tc_edge_mlp(s, bm1, wm2, bm2):
    rows = 2048
    grid = EPAD // rows
    return pl.pallas_call(
        _edge_mlp_body,
        grid=(grid,),
        in_specs=[
            pl.BlockSpec((rows, 128), lambda i: (i, 0)),
            pl.BlockSpec((1, 128), lambda i: (0, 0)),
            pl.BlockSpec((128, 16), lambda i: (0, 0)),
            pl.BlockSpec((1, 16), lambda i: (0, 0)),
        ],
        out_specs=pl.BlockSpec((rows, 16), lambda i: (i, 0)),
        out_shape=jax.ShapeDtypeStruct((EPAD, 16), _F32),
    )(s, bm1, wm2, bm2)


def kernel(x, edge_index, W1, b1, W2, b2, Wm1, bm1, Wm2, bm2):
    xp = jnp.pad(x, ((0, NPAD - N), (0, 0)))
    pad = jnp.full((EPAD - E,), N, jnp.int32)
    srcp = jnp.concatenate([edge_index[0], pad]).reshape(NW, NCHUNK, CH)
    dstp = jnp.concatenate([edge_index[1], pad]).reshape(NW, NCHUNK, CH)

    h1 = _tc_matmul(xp, W1)
    deg = _deg_call(dstp)
    g1, dinv = _tc_scale(h1, deg[0].reshape(NPAD, 1), deg[1].reshape(NPAD, 1))
    agg1 = _agg_call(g1, srcp, dstp)
    g2 = _tc_layer(agg1, g1, dinv, b1.reshape(1, 128), W2)
    agg2 = _agg_call(g2, srcp, dstp)
    A, B = _tc_final_node(agg2, g2, dinv, b2.reshape(1, 128),
                          Wm1[:128], Wm1[128:])
    S = _edge_call(A, B, srcp, dstp)
    pred = _tc_edge_mlp(S, bm1.reshape(1, 128), Wm2, bm2.reshape(1, 16))
    return pred[:E]


# speedup vs baseline: 10.9104x; 1.7391x over previous
import jax
import jax.numpy as jnp
from jax import lax
from jax.experimental import pallas as pl
from jax.experimental.pallas import tpu as pltpu
from jax.experimental.pallas import tpu_sc as plsc

N = 10000
E = 320000
NPAD = 10240
NC, NS, L = 2, 16, 16
CH = 128
C_CHUNKS = E // CH
CB = 2496 // 8
B0 = 96
TAIL0, TAILC = 2496, 4
GSZ = 40
WINC = 2456
RPT = NPAD // NS

_MESH = plsc.VectorSubcoreMesh(core_axis_name="c", subcore_axis_name="s",
                               num_cores=NC, num_subcores=NS)

_F32 = jnp.float32


def _chunk_range(cid, sid):
    base = jnp.where(cid == 0, 0, B0)
    wb = jnp.where(cid == 0, B0, CB - B0)
    t0 = 8 * (base + (sid * wb) // NS)
    t1 = 8 * (base + ((sid + 1) * wb) // NS)
    return t0, t1


def _is_tail_tile(cid, sid):
    return jnp.logical_and(cid == 1, sid == NS - 1)


def _group_geom(gi, t0, t1):
    gstart = t0 + gi * GSZ
    gcount = jnp.minimum(GSZ, t1 - gstart)
    win = pl.multiple_of(jnp.minimum(gstart, WINC), 8)
    roff = gstart - win
    return gstart, gcount, win, roff


def _ngroups(t0, t1):
    return (t1 - t0 + GSZ - 1) // GSZ


def _deg_body(dst_hbm, out_hbm, idx_v, ones_v, zb_v, acc_sh, sem):
    cid = lax.axis_index("c")
    sid = lax.axis_index("s")
    t0, t1 = _chunk_range(cid, sid)

    def fill_ones(i, c):
        ones_v[pl.ds(i * L, L)] = jnp.full((L,), 1.0, _F32)
        return c

    lax.fori_loop(0, CH // L, fill_ones, 0)

    def fill_zero(i, c):
        zb_v[pl.ds(i * L, L)] = jnp.zeros((L,), _F32)
        return c

    lax.fori_loop(0, RPT // L, fill_zero, 0)
    pltpu.sync_copy(zb_v, acc_sh.at[pl.ds(sid * RPT, RPT)])
    plsc.subcore_barrier()

    def group(gi, c):
        _, gcount, win, roff = _group_geom(gi, t0, t1)
        pltpu.async_copy(dst_hbm.at[pl.ds(win, GSZ)], idx_v, sem).wait()

        def body(j, c2):
            pltpu.sync_copy(ones_v, acc_sh.at[idx_v.at[roff + j]], add=True)
            return c2

        lax.fori_loop(0, gcount, body, 0)
        return c

    lax.fori_loop(0, _ngroups(t0, t1), group, 0)

    @pl.when(_is_tail_tile(cid, sid))
    def _():
        pltpu.async_copy(dst_hbm.at[pl.ds(TAIL0, TAILC)],
                         idx_v.at[pl.ds(0, TAILC)], sem).wait()

        def tbody(j, c):
            pltpu.sync_copy(ones_v, acc_sh.at[idx_v.at[j]], add=True)
            return c

        lax.fori_loop(0, TAILC, tbody, 0)

    plsc.subcore_barrier()
    pltpu.sync_copy(acc_sh.at[pl.ds(sid * RPT, RPT)],
                    out_hbm.at[cid, pl.ds(sid * RPT, RPT)])


_deg_call = pl.kernel(
    _deg_body,
    out_type=jax.ShapeDtypeStruct((NC, NPAD), _F32),
    mesh=_MESH,
    scratch_types=[
        pltpu.VMEM((GSZ, CH), jnp.int32),
        pltpu.VMEM((CH,), _F32),
        pltpu.VMEM((RPT,), _F32),
        pltpu.VMEM_SHARED((NPAD,), _F32),
        pltpu.SemaphoreType.DMA,
    ],
)


def _agg_body(g_hbm, src_hbm, dst_hbm, out_hbm, sidx, didx, buf, acc_sh, sem):
    cid = lax.axis_index("c")
    sid = lax.axis_index("s")
    t0, t1 = _chunk_range(cid, sid)

    def fill_zero(i, c):
        r = i // 8
        col = i % 8
        buf[0, r, pl.ds(col * L, L)] = jnp.zeros((L,), _F32)
        return c

    lax.fori_loop(0, CH * 8, fill_zero, 0)
    for t in range(RPT // CH):
        pltpu.sync_copy(buf.at[0], acc_sh.at[pl.ds(sid * RPT + t * CH, CH)])
    plsc.subcore_barrier()

    def group(gi, c):
        _, gcount, win, roff = _group_geom(gi, t0, t1)
        pltpu.async_copy(src_hbm.at[pl.ds(win, GSZ)], sidx, sem).wait()
        pltpu.async_copy(dst_hbm.at[pl.ds(win, GSZ)], didx, sem).wait()

        pltpu.async_copy(g_hbm.at[sidx.at[roff]], buf.at[0], sem)

        def body(j, c2):
            b = j % 2
            pltpu.make_async_copy(g_hbm.at[sidx.at[roff + j]], buf.at[b],
                                  sem).wait()

            @pl.when(j + 1 < gcount)
            def _():
                pltpu.async_copy(g_hbm.at[sidx.at[roff + j + 1]],
                                 buf.at[(j + 1) % 2], sem)

            pltpu.sync_copy(buf.at[b], acc_sh.at[didx.at[roff + j]], add=True)
            return c2

        lax.fori_loop(0, gcount, body, 0)
        return c

    lax.fori_loop(0, _ngroups(t0, t1), group, 0)

    @pl.when(_is_tail_tile(cid, sid))
    def _():
        pltpu.async_copy(src_hbm.at[pl.ds(TAIL0, TAILC)],
                         sidx.at[pl.ds(0, TAILC)], sem).wait()
        pltpu.async_copy(dst_hbm.at[pl.ds(TAIL0, TAILC)],
                         didx.at[pl.ds(0, TAILC)], sem).wait()

        def tbody(j, c):
            pltpu.async_copy(g_hbm.at[sidx.at[j]], buf.at[0], sem).wait()
            pltpu.sync_copy(buf.at[0], acc_sh.at[didx.at[j]], add=True)
            return c

        lax.fori_loop(0, TAILC, tbody, 0)

    plsc.subcore_barrier()
    pltpu.sync_copy(acc_sh.at[pl.ds(sid * RPT, RPT)],
                    out_hbm.at[cid, pl.ds(sid * RPT, RPT)])


_agg_call = pl.kernel(
    _agg_body,
    out_type=jax.ShapeDtypeStruct((NC, NPAD, 128), _F32),
    mesh=_MESH,
    scratch_types=[
        pltpu.VMEM((GSZ, CH), jnp.int32),
        pltpu.VMEM((GSZ, CH), jnp.int32),
        pltpu.VMEM((2, CH, 128), _F32),
        pltpu.VMEM_SHARED((NPAD, 128), _F32),
        pltpu.SemaphoreType.DMA,
    ],
)


def _edge_body(a_hbm, b_hbm, src_hbm, dst_hbm, out_hbm, sidx, didx, buf,
               sem_a, sem_b, sem_o):
    cid = lax.axis_index("c")
    sid = lax.axis_index("s")
    t0, t1 = _chunk_range(cid, sid)

    def group(gi, c):
        gstart, gcount, win, roff = _group_geom(gi, t0, t1)
        pltpu.async_copy(src_hbm.at[pl.ds(win, GSZ)], sidx, sem_a).wait()
        pltpu.async_copy(dst_hbm.at[pl.ds(win, GSZ)], didx, sem_a).wait()

        pltpu.async_copy(a_hbm.at[sidx.at[roff]], buf.at[0], sem_a)

        def body(j, c2):
            b = j % 2
            pltpu.make_async_copy(a_hbm.at[sidx.at[roff + j]], buf.at[b],
                                  sem_a).wait()
            pltpu.async_copy(b_hbm.at[didx.at[roff + j]], buf.at[b], sem_b,
                             add=True)

            @pl.when(j >= 1)
            def _():
                pltpu.make_async_copy(
                    buf.at[1 - b],
                    out_hbm.at[pl.ds(pl.multiple_of((gstart + j - 1) * CH, CH),
                                     CH)],
                    sem_o).wait()

            @pl.when(j + 1 < gcount)
            def _():
                pltpu.async_copy(a_hbm.at[sidx.at[roff + j + 1]],
                                 buf.at[1 - b], sem_a)

            pltpu.make_async_copy(b_hbm.at[didx.at[roff + j]], buf.at[b],
                                  sem_b).wait()
            pltpu.async_copy(
                buf.at[b],
                out_hbm.at[pl.ds(pl.multiple_of((gstart + j) * CH, CH), CH)],
                sem_o)
            return c2

        lax.fori_loop(0, gcount, body, 0)
        pltpu.make_async_copy(
            buf.at[(gcount - 1) % 2],
            out_hbm.at[pl.ds(pl.multiple_of((gstart + gcount - 1) * CH, CH),
                             CH)],
            sem_o).wait()
        return c

    lax.fori_loop(0, _ngroups(t0, t1), group, 0)

    @pl.when(_is_tail_tile(cid, sid))
    def _():
        pltpu.async_copy(src_hbm.at[pl.ds(TAIL0, TAILC)],
                         sidx.at[pl.ds(0, TAILC)], sem_a).wait()
        pltpu.async_copy(dst_hbm.at[pl.ds(TAIL0, TAILC)],
                         didx.at[pl.ds(0, TAILC)], sem_a).wait()

        def tbody(j, c):
            pltpu.async_copy(a_hbm.at[sidx.at[j]], buf.at[0], sem_a).wait()
            pltpu.async_copy(b_hbm.at[didx.at[j]], buf.at[0], sem_b,
                             add=True).wait()
            pltpu.sync_copy(
                buf.at[0],
                out_hbm.at[pl.ds(pl.multiple_of((TAIL0 + j) * CH, CH), CH)])
            return c

        lax.fori_loop(0, TAILC, tbody, 0)


_edge_call = pl.kernel(
    _edge_body,
    out_type=jax.ShapeDtypeStruct((E, 128), _F32),
    mesh=_MESH,
    scratch_types=[
        pltpu.VMEM((GSZ, CH), jnp.int32),
        pltpu.VMEM((GSZ, CH), jnp.int32),
        pltpu.VMEM((2, CH, 128), _F32),
        pltpu.SemaphoreType.DMA,
        pltpu.SemaphoreType.DMA,
        pltpu.SemaphoreType.DMA,
    ],
)


_PREC = lax.Precision.HIGHEST


def _mm_body(x_ref, w_ref, o_ref):
    o_ref[:] = jnp.dot(x_ref[:], w_ref[:], preferred_element_type=_F32,
                       precision=_PREC)


def _tc_matmul(x, w, rows_per_block=1280):
    m = x.shape[0]
    grid = m // rows_per_block
    return pl.pallas_call(
        _mm_body,
        grid=(grid,),
        in_specs=[
            pl.BlockSpec((rows_per_block, x.shape[1]), lambda i: (i, 0)),
            pl.BlockSpec(w.shape, lambda i: (0, 0)),
        ],
        out_specs=pl.BlockSpec((rows_per_block, w.shape[1]), lambda i: (i, 0)),
        out_shape=jax.ShapeDtypeStruct((m, w.shape[1]), _F32),
    )(x, w)


def _scale_body(h_ref, d0_ref, d1_ref, g_ref, dinv_ref):
    dinv = lax.rsqrt(d0_ref[:] + d1_ref[:] + 1.0)
    dinv_ref[:] = dinv
    g_ref[:] = h_ref[:] * dinv


def _tc_scale(h, d0, d1):
    grid = NPAD // 1280
    return pl.pallas_call(
        _scale_body,
        grid=(grid,),
        in_specs=[
            pl.BlockSpec((1280, 128), lambda i: (i, 0)),
            pl.BlockSpec((1280, 1), lambda i: (i, 0)),
            pl.BlockSpec((1280, 1), lambda i: (i, 0)),
        ],
        out_specs=[
            pl.BlockSpec((1280, 128), lambda i: (i, 0)),
            pl.BlockSpec((1280, 1), lambda i: (i, 0)),
        ],
        out_shape=[
            jax.ShapeDtypeStruct((NPAD, 128), _F32),
            jax.ShapeDtypeStruct((NPAD, 1), _F32),
        ],
    )(h, d0, d1)


def _layer_body(a0_ref, a1_ref, g_ref, dinv_ref, b_ref, w_ref, o_ref):
    dinv = dinv_ref[:]
    h = (a0_ref[:] + a1_ref[:] + g_ref[:]) * dinv + b_ref[:]
    h = jnp.maximum(h, 0.0)
    o_ref[:] = jnp.dot(h, w_ref[:], preferred_element_type=_F32,
                       precision=_PREC) * dinv


def _tc_layer(agg, g, dinv, b, w):
    grid = NPAD // 1280
    return pl.pallas_call(
        _layer_body,
        grid=(grid,),
        in_specs=[
            pl.BlockSpec((1280, 128), lambda i: (i, 0)),
            pl.BlockSpec((1280, 128), lambda i: (i, 0)),
            pl.BlockSpec((1280, 128), lambda i: (i, 0)),
            pl.BlockSpec((1280, 1), lambda i: (i, 0)),
            pl.BlockSpec((1, 128), lambda i: (0, 0)),
            pl.BlockSpec((128, 128), lambda i: (0, 0)),
        ],
        out_specs=pl.BlockSpec((1280, 128), lambda i: (i, 0)),
        out_shape=jax.ShapeDtypeStruct((NPAD, 128), _F32),
    )(agg[0], agg[1], g, dinv, b, w)


def _final_node_body(a0_ref, a1_ref, g_ref, dinv_ref, b_ref, wa_ref, wb_ref,
                     oa_ref, ob_ref):
    dinv = dinv_ref[:]
    h = (a0_ref[:] + a1_ref[:] + g_ref[:]) * dinv + b_ref[:]
    h = jnp.maximum(h, 0.0)
    oa_ref[:] = jnp.dot(h, wa_ref[:], preferred_element_type=_F32,
                        precision=_PREC)
    ob_ref[:] = jnp.dot(h, wb_ref[:], preferred_element_type=_F32,
                        precision=_PREC)


def _tc_final_node(agg, g, dinv, b, wa, wb):
    grid = NPAD // 1280
    return pl.pallas_call(
        _final_node_body,
        grid=(grid,),
        in_specs=[
            pl.BlockSpec((1280, 128), lambda i: (i, 0)),
            pl.BlockSpec((1280, 128), lambda i: (i, 0)),
            pl.BlockSpec((1280, 128), lambda i: (i, 0)),
            pl.BlockSpec((1280, 1), lambda i: (i, 0)),
            pl.BlockSpec((1, 128), lambda i: (0, 0)),
            pl.BlockSpec((128, 128), lambda i: (0, 0)),
            pl.BlockSpec((128, 128), lambda i: (0, 0)),
        ],
        out_specs=[
            pl.BlockSpec((1280, 128), lambda i: (i, 0)),
            pl.BlockSpec((1280, 128), lambda i: (i, 0)),
        ],
        out_shape=[
            jax.ShapeDtypeStruct((NPAD, 128), _F32),
            jax.ShapeDtypeStruct((NPAD, 128), _F32),
        ],
    )(agg[0], agg[1], g, dinv, b, wa, wb)


def _edge_mlp_body(s_ref, b1_ref, w2_ref, b2_ref, o_ref):
    z = jnp.maximum(s_ref[:] + b1_ref[:], 0.0)
    o_ref[:] = jnp.dot(z, w2_ref[:], preferred_element_type=_F32,
                       precision=_PREC) + b2_ref[:]


def _tc_edge_mlp(s, bm1, wm2, bm2):
    rows = 2000
    grid = E // rows
    return pl.pallas_call(
        _edge_mlp_body,
        grid=(grid,),
        in_specs=[
            pl.BlockSpec((rows, 128), lambda i: (i, 0)),
            pl.BlockSpec((1, 128), lambda i: (0, 0)),
            pl.BlockSpec((128, 16), lambda i: (0, 0)),
            pl.BlockSpec((1, 16), lambda i: (0, 0)),
        ],
        out_specs=pl.BlockSpec((rows, 16), lambda i: (i, 0)),
        out_shape=jax.ShapeDtypeStruct((E, 16), _F32),
    )(s, bm1, wm2, bm2)


def kernel(x, edge_index, W1, b1, W2, b2, Wm1, bm1, Wm2, bm2):
    xp = jnp.pad(x, ((0, NPAD - N), (0, 0)))
    srcc = edge_index[0].reshape(C_CHUNKS, CH)
    dstc = edge_index[1].reshape(C_CHUNKS, CH)

    h1 = _tc_matmul(xp, W1)
    deg = _deg_call(dstc)
    g1, dinv = _tc_scale(h1, deg[0].reshape(NPAD, 1), deg[1].reshape(NPAD, 1))
    agg1 = _agg_call(g1, srcc, dstc)
    g2 = _tc_layer(agg1, g1, dinv, b1.reshape(1, 128), W2)
    agg2 = _agg_call(g2, srcc, dstc)
    A, B = _tc_final_node(agg2, g2, dinv, b2.reshape(1, 128),
                          Wm1[:128], Wm1[128:])
    S = _edge_call(A, B, srcc, dstc)
    pred = _tc_edge_mlp(S, bm1.reshape(1, 128), Wm2, bm2.reshape(1, 16))
    return pred


# speedup vs baseline: 14.0192x; 1.2849x over previous
import jax
import jax.numpy as jnp
from jax import lax
from jax.experimental import pallas as pl
from jax.experimental.pallas import tpu as pltpu
from jax.experimental.pallas import tpu_sc as plsc

N = 10000
E = 320000
NPAD = 10240
NC, NS, L = 2, 16, 16
CH = 128
C_CHUNKS = E // CH
CB = 2496 // 8
B0 = 156
TAIL0, TAILC = 2496, 4
GSZ = 40
WINC = 2456
RPT = NPAD // NS

_MESH = plsc.VectorSubcoreMesh(core_axis_name="c", subcore_axis_name="s",
                               num_cores=NC, num_subcores=NS)

_F32 = jnp.float32


def _chunk_range(cid, sid):
    base = jnp.where(cid == 0, 0, B0)
    wb = jnp.where(cid == 0, B0, CB - B0)
    t0 = 8 * (base + (sid * wb) // NS)
    t1 = 8 * (base + ((sid + 1) * wb) // NS)
    return t0, t1


def _is_tail_tile(cid, sid):
    return jnp.logical_and(cid == 1, sid == NS - 1)


def _group_geom(gi, t0, t1):
    gstart = t0 + gi * GSZ
    gcount = jnp.minimum(GSZ, t1 - gstart)
    win = pl.multiple_of(jnp.minimum(gstart, WINC), 8)
    roff = gstart - win
    return gstart, gcount, win, roff


def _ngroups(t0, t1):
    return (t1 - t0 + GSZ - 1) // GSZ


def _deg_body(dst_hbm, out_hbm, idx_v, ones_v, zb_v, acc_sh, sem):
    cid = lax.axis_index("c")
    sid = lax.axis_index("s")
    t0, t1 = _chunk_range(cid, sid)

    def fill_ones(i, c):
        ones_v[pl.ds(i * L, L)] = jnp.full((L,), 1.0, _F32)
        return c

    lax.fori_loop(0, CH // L, fill_ones, 0)

    def fill_zero(i, c):
        zb_v[pl.ds(i * L, L)] = jnp.zeros((L,), _F32)
        return c

    lax.fori_loop(0, RPT // L, fill_zero, 0)
    pltpu.sync_copy(zb_v, acc_sh.at[pl.ds(sid * RPT, RPT)])
    plsc.subcore_barrier()

    def group(gi, c):
        _, gcount, win, roff = _group_geom(gi, t0, t1)
        pltpu.async_copy(dst_hbm.at[pl.ds(win, GSZ)], idx_v, sem).wait()

        def body(j, c2):
            pltpu.sync_copy(ones_v, acc_sh.at[idx_v.at[roff + j]], add=True)
            return c2

        lax.fori_loop(0, gcount, body, 0)
        return c

    lax.fori_loop(0, _ngroups(t0, t1), group, 0)

    @pl.when(_is_tail_tile(cid, sid))
    def _():
        pltpu.async_copy(dst_hbm.at[pl.ds(TAIL0, TAILC)],
                         idx_v.at[pl.ds(0, TAILC)], sem).wait()

        def tbody(j, c):
            pltpu.sync_copy(ones_v, acc_sh.at[idx_v.at[j]], add=True)
            return c

        lax.fori_loop(0, TAILC, tbody, 0)

    plsc.subcore_barrier()
    pltpu.sync_copy(acc_sh.at[pl.ds(sid * RPT, RPT)],
                    out_hbm.at[cid, pl.ds(sid * RPT, RPT)])


_deg_call = pl.kernel(
    _deg_body,
    out_type=jax.ShapeDtypeStruct((NC, NPAD), _F32),
    mesh=_MESH,
    scratch_types=[
        pltpu.VMEM((GSZ, CH), jnp.int32),
        pltpu.VMEM((CH,), _F32),
        pltpu.VMEM((RPT,), _F32),
        pltpu.VMEM_SHARED((NPAD,), _F32),
        pltpu.SemaphoreType.DMA,
    ],
)


def _agg_body(g_hbm, src_hbm, dst_hbm, out_hbm, sidx, didx, buf, acc_sh, sem):
    cid = lax.axis_index("c")
    sid = lax.axis_index("s")
    t0, t1 = _chunk_range(cid, sid)

    def fill_zero(i, c):
        r = i // 8
        col = i % 8
        buf[0, r, pl.ds(col * L, L)] = jnp.zeros((L,), _F32)
        return c

    lax.fori_loop(0, CH * 8, fill_zero, 0)
    for t in range(RPT // CH):
        pltpu.sync_copy(buf.at[0], acc_sh.at[pl.ds(sid * RPT + t * CH, CH)])
    plsc.subcore_barrier()

    def group(gi, c):
        _, gcount, win, roff = _group_geom(gi, t0, t1)
        pltpu.async_copy(src_hbm.at[pl.ds(win, GSZ)], sidx, sem).wait()
        pltpu.async_copy(dst_hbm.at[pl.ds(win, GSZ)], didx, sem).wait()

        pltpu.async_copy(g_hbm.at[sidx.at[roff]], buf.at[0], sem)

        def body(j, c2):
            b = j % 2
            pltpu.make_async_copy(g_hbm.at[sidx.at[roff + j]], buf.at[b],
                                  sem).wait()

            @pl.when(j + 1 < gcount)
            def _():
                pltpu.async_copy(g_hbm.at[sidx.at[roff + j + 1]],
                                 buf.at[(j + 1) % 2], sem)

            pltpu.sync_copy(buf.at[b], acc_sh.at[didx.at[roff + j]], add=True)
            return c2

        lax.fori_loop(0, gcount, body, 0)
        return c

    lax.fori_loop(0, _ngroups(t0, t1), group, 0)

    @pl.when(_is_tail_tile(cid, sid))
    def _():
        pltpu.async_copy(src_hbm.at[pl.ds(TAIL0, TAILC)],
                         sidx.at[pl.ds(0, TAILC)], sem).wait()
        pltpu.async_copy(dst_hbm.at[pl.ds(TAIL0, TAILC)],
                         didx.at[pl.ds(0, TAILC)], sem).wait()

        def tbody(j, c):
            pltpu.async_copy(g_hbm.at[sidx.at[j]], buf.at[0], sem).wait()
            pltpu.sync_copy(buf.at[0], acc_sh.at[didx.at[j]], add=True)
            return c

        lax.fori_loop(0, TAILC, tbody, 0)

    plsc.subcore_barrier()
    pltpu.sync_copy(acc_sh.at[pl.ds(sid * RPT, RPT)],
                    out_hbm.at[cid, pl.ds(sid * RPT, RPT)])


_agg_call = pl.kernel(
    _agg_body,
    out_type=jax.ShapeDtypeStruct((NC, NPAD, 128), _F32),
    mesh=_MESH,
    scratch_types=[
        pltpu.VMEM((GSZ, CH), jnp.int32),
        pltpu.VMEM((GSZ, CH), jnp.int32),
        pltpu.VMEM((2, CH, 128), _F32),
        pltpu.VMEM_SHARED((NPAD, 128), _F32),
        pltpu.SemaphoreType.DMA,
    ],
)


def _edge_body(a_hbm, b_hbm, src_hbm, dst_hbm, out_hbm, sidx, didx, buf,
               sem_a, sem_b, sem_o):
    cid = lax.axis_index("c")
    sid = lax.axis_index("s")
    t0, t1 = _chunk_range(cid, sid)

    def group(gi, c):
        gstart, gcount, win, roff = _group_geom(gi, t0, t1)
        pltpu.async_copy(src_hbm.at[pl.ds(win, GSZ)], sidx, sem_a).wait()
        pltpu.async_copy(dst_hbm.at[pl.ds(win, GSZ)], didx, sem_a).wait()

        pltpu.async_copy(a_hbm.at[sidx.at[roff]], buf.at[0], sem_a)

        def body(j, c2):
            b = j % 2
            pltpu.make_async_copy(a_hbm.at[sidx.at[roff + j]], buf.at[b],
                                  sem_a).wait()
            pltpu.async_copy(b_hbm.at[didx.at[roff + j]], buf.at[b], sem_b,
                             add=True)

            @pl.when(j >= 1)
            def _():
                pltpu.make_async_copy(
                    buf.at[1 - b],
                    out_hbm.at[pl.ds(pl.multiple_of((gstart + j - 1) * CH, CH),
                                     CH)],
                    sem_o).wait()

            @pl.when(j + 1 < gcount)
            def _():
                pltpu.async_copy(a_hbm.at[sidx.at[roff + j + 1]],
                                 buf.at[1 - b], sem_a)

            pltpu.make_async_copy(b_hbm.at[didx.at[roff + j]], buf.at[b],
                                  sem_b).wait()
            pltpu.async_copy(
                buf.at[b],
                out_hbm.at[pl.ds(pl.multiple_of((gstart + j) * CH, CH), CH)],
                sem_o)
            return c2

        lax.fori_loop(0, gcount, body, 0)
        pltpu.make_async_copy(
            buf.at[(gcount - 1) % 2],
            out_hbm.at[pl.ds(pl.multiple_of((gstart + gcount - 1) * CH, CH),
                             CH)],
            sem_o).wait()
        return c

    lax.fori_loop(0, _ngroups(t0, t1), group, 0)

    @pl.when(_is_tail_tile(cid, sid))
    def _():
        pltpu.async_copy(src_hbm.at[pl.ds(TAIL0, TAILC)],
                         sidx.at[pl.ds(0, TAILC)], sem_a).wait()
        pltpu.async_copy(dst_hbm.at[pl.ds(TAIL0, TAILC)],
                         didx.at[pl.ds(0, TAILC)], sem_a).wait()

        def tbody(j, c):
            pltpu.async_copy(a_hbm.at[sidx.at[j]], buf.at[0], sem_a).wait()
            pltpu.async_copy(b_hbm.at[didx.at[j]], buf.at[0], sem_b,
                             add=True).wait()
            pltpu.sync_copy(
                buf.at[0],
                out_hbm.at[pl.ds(pl.multiple_of((TAIL0 + j) * CH, CH), CH)])
            return c

        lax.fori_loop(0, TAILC, tbody, 0)


_edge_call = pl.kernel(
    _edge_body,
    out_type=jax.ShapeDtypeStruct((E, 128), _F32),
    mesh=_MESH,
    scratch_types=[
        pltpu.VMEM((GSZ, CH), jnp.int32),
        pltpu.VMEM((GSZ, CH), jnp.int32),
        pltpu.VMEM((2, CH, 128), _F32),
        pltpu.SemaphoreType.DMA,
        pltpu.SemaphoreType.DMA,
        pltpu.SemaphoreType.DMA,
    ],
)


_PREC = lax.Precision.HIGHEST


def _mm_body(x_ref, w_ref, o_ref):
    o_ref[:] = jnp.dot(x_ref[:], w_ref[:], preferred_element_type=_F32,
                       precision=_PREC)


def _tc_matmul(x, w, rows_per_block=1280):
    m = x.shape[0]
    grid = m // rows_per_block
    return pl.pallas_call(
        _mm_body,
        grid=(grid,),
        in_specs=[
            pl.BlockSpec((rows_per_block, x.shape[1]), lambda i: (i, 0)),
            pl.BlockSpec(w.shape, lambda i: (0, 0)),
        ],
        out_specs=pl.BlockSpec((rows_per_block, w.shape[1]), lambda i: (i, 0)),
        out_shape=jax.ShapeDtypeStruct((m, w.shape[1]), _F32),
    )(x, w)


def _scale_body(h_ref, d_ref, g_ref, dinv_ref):
    dinv = lax.rsqrt(d_ref[0] + d_ref[1] + 1.0)
    dinv_ref[:] = dinv
    g_ref[:] = h_ref[:] * dinv


def _tc_scale(h, deg):
    grid = NPAD // 1280
    return pl.pallas_call(
        _scale_body,
        grid=(grid,),
        in_specs=[
            pl.BlockSpec((1280, 128), lambda i: (i, 0)),
            pl.BlockSpec((2, 1280, 1), lambda i: (0, i, 0)),
        ],
        out_specs=[
            pl.BlockSpec((1280, 128), lambda i: (i, 0)),
            pl.BlockSpec((1280, 1), lambda i: (i, 0)),
        ],
        out_shape=[
            jax.ShapeDtypeStruct((NPAD, 128), _F32),
            jax.ShapeDtypeStruct((NPAD, 1), _F32),
        ],
    )(h, deg)


def _layer_body(a_ref, g_ref, dinv_ref, b_ref, w_ref, o_ref):
    dinv = dinv_ref[:]
    h = (a_ref[0] + a_ref[1] + g_ref[:]) * dinv + b_ref[:]
    h = jnp.maximum(h, 0.0)
    o_ref[:] = jnp.dot(h, w_ref[:], preferred_element_type=_F32,
                       precision=_PREC) * dinv


def _tc_layer(agg, g, dinv, b, w):
    grid = NPAD // 1280
    return pl.pallas_call(
        _layer_body,
        grid=(grid,),
        in_specs=[
            pl.BlockSpec((2, 1280, 128), lambda i: (0, i, 0)),
            pl.BlockSpec((1280, 128), lambda i: (i, 0)),
            pl.BlockSpec((1280, 1), lambda i: (i, 0)),
            pl.BlockSpec((1, 128), lambda i: (0, 0)),
            pl.BlockSpec((128, 128), lambda i: (0, 0)),
        ],
        out_specs=pl.BlockSpec((1280, 128), lambda i: (i, 0)),
        out_shape=jax.ShapeDtypeStruct((NPAD, 128), _F32),
    )(agg, g, dinv, b, w)


def _final_node_body(a_ref, g_ref, dinv_ref, b_ref, wa_ref, wb_ref,
                     oa_ref, ob_ref):
    dinv = dinv_ref[:]
    h = (a_ref[0] + a_ref[1] + g_ref[:]) * dinv + b_ref[:]
    h = jnp.maximum(h, 0.0)
    oa_ref[:] = jnp.dot(h, wa_ref[:], preferred_element_type=_F32,
                        precision=_PREC)
    ob_ref[:] = jnp.dot(h, wb_ref[:], preferred_element_type=_F32,
                        precision=_PREC)


def _tc_final_node(agg, g, dinv, b, wa, wb):
    grid = NPAD // 1280
    return pl.pallas_call(
        _final_node_body,
        grid=(grid,),
        in_specs=[
            pl.BlockSpec((2, 1280, 128), lambda i: (0, i, 0)),
            pl.BlockSpec((1280, 128), lambda i: (i, 0)),
            pl.BlockSpec((1280, 1), lambda i: (i, 0)),
            pl.BlockSpec((1, 128), lambda i: (0, 0)),
            pl.BlockSpec((128, 128), lambda i: (0, 0)),
            pl.BlockSpec((128, 128), lambda i: (0, 0)),
        ],
        out_specs=[
            pl.BlockSpec((1280, 128), lambda i: (i, 0)),
            pl.BlockSpec((1280, 128), lambda i: (i, 0)),
        ],
        out_shape=[
            jax.ShapeDtypeStruct((NPAD, 128), _F32),
            jax.ShapeDtypeStruct((NPAD, 128), _F32),
        ],
    )(agg, g, dinv, b, wa, wb)


def _edge_mlp_body(s_ref, b1_ref, w2_ref, b2_ref, o_ref):
    z = jnp.maximum(s_ref[:] + b1_ref[:], 0.0)
    o_ref[:] = jnp.dot(z, w2_ref[:], preferred_element_type=_F32,
                       precision=_PREC) + b2_ref[:]


def _tc_edge_mlp(s, bm1, wm2, bm2):
    rows = 8000
    grid = E // rows
    return pl.pallas_call(
        _edge_mlp_body,
        grid=(grid,),
        in_specs=[
            pl.BlockSpec((rows, 128), lambda i: (i, 0)),
            pl.BlockSpec((1, 128), lambda i: (0, 0)),
            pl.BlockSpec((128, 16), lambda i: (0, 0)),
            pl.BlockSpec((1, 16), lambda i: (0, 0)),
        ],
        out_specs=pl.BlockSpec((rows, 16), lambda i: (i, 0)),
        out_shape=jax.ShapeDtypeStruct((E, 16), _F32),
    )(s, bm1, wm2, bm2)


def kernel(x, edge_index, W1, b1, W2, b2, Wm1, bm1, Wm2, bm2):
    xp = jnp.pad(x, ((0, NPAD - N), (0, 0)))
    srcc = edge_index[0].reshape(C_CHUNKS, CH)
    dstc = edge_index[1].reshape(C_CHUNKS, CH)

    h1 = _tc_matmul(xp, W1)
    deg = _deg_call(dstc)
    g1, dinv = _tc_scale(h1, deg.reshape(NC, NPAD, 1))
    agg1 = _agg_call(g1, srcc, dstc)
    g2 = _tc_layer(agg1, g1, dinv, b1.reshape(1, 128), W2)
    agg2 = _agg_call(g2, srcc, dstc)
    A, B = _tc_final_node(agg2, g2, dinv, b2.reshape(1, 128),
                          Wm1[:128], Wm1[128:])
    S = _edge_call(A, B, srcc, dstc)
    pred = _tc_edge_mlp(S, bm1.reshape(1, 128), Wm2, bm2.reshape(1, 16))
    return pred


# speedup vs baseline: 16.0429x; 1.1444x over previous
import jax
import jax.numpy as jnp
from jax import lax
from jax.experimental import pallas as pl
from jax.experimental.pallas import tpu as pltpu
from jax.experimental.pallas import tpu_sc as plsc

N = 10000
E = 320000
NPAD = 10240
NC, NS, L = 2, 16, 16
CH = 128
C_CHUNKS = E // CH
CB = 2496 // 8
B0 = 156
TAIL0, TAILC = 2496, 4
GSZ = 40
WINC = 2456
RPT = NPAD // NS

_MESH = plsc.VectorSubcoreMesh(core_axis_name="c", subcore_axis_name="s",
                               num_cores=NC, num_subcores=NS)

_F32 = jnp.float32


def _chunk_range(cid, sid):
    base = jnp.where(cid == 0, 0, B0)
    wb = jnp.where(cid == 0, B0, CB - B0)
    t0 = 8 * (base + (sid * wb) // NS)
    t1 = 8 * (base + ((sid + 1) * wb) // NS)
    return t0, t1


def _is_tail_tile(cid, sid):
    return jnp.logical_and(cid == 1, sid == NS - 1)


def _group_geom(gi, t0, t1):
    gstart = t0 + gi * GSZ
    gcount = jnp.minimum(GSZ, t1 - gstart)
    win = pl.multiple_of(jnp.minimum(gstart, WINC), 8)
    roff = gstart - win
    return gstart, gcount, win, roff


def _ngroups(t0, t1):
    return (t1 - t0 + GSZ - 1) // GSZ


def _deg_body(dst_hbm, out_hbm, idx_v, ones_v, zb_v, acc_sh, sem):
    cid = lax.axis_index("c")
    sid = lax.axis_index("s")
    t0, t1 = _chunk_range(cid, sid)

    def fill_ones(i, c):
        ones_v[pl.ds(i * L, L)] = jnp.full((L,), 1.0, _F32)
        return c

    lax.fori_loop(0, CH // L, fill_ones, 0)

    def fill_zero(i, c):
        zb_v[pl.ds(i * L, L)] = jnp.zeros((L,), _F32)
        return c

    lax.fori_loop(0, RPT // L, fill_zero, 0)
    pltpu.sync_copy(zb_v, acc_sh.at[pl.ds(sid * RPT, RPT)])
    plsc.subcore_barrier()

    def group(gi, c):
        _, gcount, win, roff = _group_geom(gi, t0, t1)
        pltpu.async_copy(dst_hbm.at[pl.ds(win, GSZ)], idx_v, sem).wait()

        def body(j, c2):
            pltpu.sync_copy(ones_v, acc_sh.at[idx_v.at[roff + j]], add=True)
            return c2

        lax.fori_loop(0, gcount, body, 0)
        return c

    lax.fori_loop(0, _ngroups(t0, t1), group, 0)

    @pl.when(_is_tail_tile(cid, sid))
    def _():
        pltpu.async_copy(dst_hbm.at[pl.ds(TAIL0, TAILC)],
                         idx_v.at[pl.ds(0, TAILC)], sem).wait()

        def tbody(j, c):
            pltpu.sync_copy(ones_v, acc_sh.at[idx_v.at[j]], add=True)
            return c

        lax.fori_loop(0, TAILC, tbody, 0)

    plsc.subcore_barrier()
    pltpu.sync_copy(acc_sh.at[pl.ds(sid * RPT, RPT)],
                    out_hbm.at[cid, pl.ds(sid * RPT, RPT)])


_deg_call = pl.kernel(
    _deg_body,
    out_type=jax.ShapeDtypeStruct((NC, NPAD), _F32),
    mesh=_MESH,
    scratch_types=[
        pltpu.VMEM((GSZ, CH), jnp.int32),
        pltpu.VMEM((CH,), _F32),
        pltpu.VMEM((RPT,), _F32),
        pltpu.VMEM_SHARED((NPAD,), _F32),
        pltpu.SemaphoreType.DMA,
    ],
)


def _agg_body(g_hbm, src_hbm, dst_hbm, out_hbm, sidx, didx, buf, acc_sh, sem):
    cid = lax.axis_index("c")
    sid = lax.axis_index("s")
    t0, t1 = _chunk_range(cid, sid)

    def fill_zero(i, c):
        r = i // 8
        col = i % 8
        buf[0, r, pl.ds(col * L, L)] = jnp.zeros((L,), _F32)
        return c

    lax.fori_loop(0, CH * 8, fill_zero, 0)
    for t in range(RPT // CH):
        pltpu.sync_copy(buf.at[0], acc_sh.at[pl.ds(sid * RPT + t * CH, CH)])
    plsc.subcore_barrier()

    def group(gi, c):
        _, gcount, win, roff = _group_geom(gi, t0, t1)
        pltpu.async_copy(src_hbm.at[pl.ds(win, GSZ)], sidx, sem).wait()
        pltpu.async_copy(dst_hbm.at[pl.ds(win, GSZ)], didx, sem).wait()

        pltpu.async_copy(g_hbm.at[sidx.at[roff]], buf.at[0], sem)

        def body(j, c2):
            b = j % 2
            pltpu.make_async_copy(g_hbm.at[sidx.at[roff + j]], buf.at[b],
                                  sem).wait()

            @pl.when(j + 1 < gcount)
            def _():
                pltpu.async_copy(g_hbm.at[sidx.at[roff + j + 1]],
                                 buf.at[(j + 1) % 2], sem)

            pltpu.sync_copy(buf.at[b], acc_sh.at[didx.at[roff + j]], add=True)
            return c2

        lax.fori_loop(0, gcount, body, 0)
        return c

    lax.fori_loop(0, _ngroups(t0, t1), group, 0)

    @pl.when(_is_tail_tile(cid, sid))
    def _():
        pltpu.async_copy(src_hbm.at[pl.ds(TAIL0, TAILC)],
                         sidx.at[pl.ds(0, TAILC)], sem).wait()
        pltpu.async_copy(dst_hbm.at[pl.ds(TAIL0, TAILC)],
                         didx.at[pl.ds(0, TAILC)], sem).wait()

        def tbody(j, c):
            pltpu.async_copy(g_hbm.at[sidx.at[j]], buf.at[0], sem).wait()
            pltpu.sync_copy(buf.at[0], acc_sh.at[didx.at[j]], add=True)
            return c

        lax.fori_loop(0, TAILC, tbody, 0)

    plsc.subcore_barrier()
    pltpu.sync_copy(acc_sh.at[pl.ds(sid * RPT, RPT)],
                    out_hbm.at[cid, pl.ds(sid * RPT, RPT)])


_agg_call = pl.kernel(
    _agg_body,
    out_type=jax.ShapeDtypeStruct((NC, NPAD, 128), _F32),
    mesh=_MESH,
    scratch_types=[
        pltpu.VMEM((GSZ, CH), jnp.int32),
        pltpu.VMEM((GSZ, CH), jnp.int32),
        pltpu.VMEM((2, CH, 128), _F32),
        pltpu.VMEM_SHARED((NPAD, 128), _F32),
        pltpu.SemaphoreType.DMA,
    ],
)


def _edge_body(a_hbm, b_hbm, src_hbm, dst_hbm, out_hbm, sidx, didx, buf,
               sem_a, sem_b, sem_o):
    cid = lax.axis_index("c")
    sid = lax.axis_index("s")
    t0, t1 = _chunk_range(cid, sid)

    def group(gi, c):
        gstart, gcount, win, roff = _group_geom(gi, t0, t1)
        pltpu.async_copy(src_hbm.at[pl.ds(win, GSZ)], sidx, sem_a).wait()
        pltpu.async_copy(dst_hbm.at[pl.ds(win, GSZ)], didx, sem_a).wait()

        pltpu.async_copy(a_hbm.at[sidx.at[roff]], buf.at[0], sem_a)

        def body(j, c2):
            b = j % 2
            pltpu.make_async_copy(a_hbm.at[sidx.at[roff + j]], buf.at[b],
                                  sem_a).wait()
            pltpu.async_copy(b_hbm.at[didx.at[roff + j]], buf.at[b], sem_b,
                             add=True)

            @pl.when(j >= 1)
            def _():
                pltpu.make_async_copy(
                    buf.at[1 - b],
                    out_hbm.at[pl.ds(pl.multiple_of((gstart + j - 1) * CH, CH),
                                     CH)],
                    sem_o).wait()

            @pl.when(j + 1 < gcount)
            def _():
                pltpu.async_copy(a_hbm.at[sidx.at[roff + j + 1]],
                                 buf.at[1 - b], sem_a)

            pltpu.make_async_copy(b_hbm.at[didx.at[roff + j]], buf.at[b],
                                  sem_b).wait()
            pltpu.async_copy(
                buf.at[b],
                out_hbm.at[pl.ds(pl.multiple_of((gstart + j) * CH, CH), CH)],
                sem_o)
            return c2

        lax.fori_loop(0, gcount, body, 0)
        pltpu.make_async_copy(
            buf.at[(gcount - 1) % 2],
            out_hbm.at[pl.ds(pl.multiple_of((gstart + gcount - 1) * CH, CH),
                             CH)],
            sem_o).wait()
        return c

    lax.fori_loop(0, _ngroups(t0, t1), group, 0)

    @pl.when(_is_tail_tile(cid, sid))
    def _():
        pltpu.async_copy(src_hbm.at[pl.ds(TAIL0, TAILC)],
                         sidx.at[pl.ds(0, TAILC)], sem_a).wait()
        pltpu.async_copy(dst_hbm.at[pl.ds(TAIL0, TAILC)],
                         didx.at[pl.ds(0, TAILC)], sem_a).wait()

        def tbody(j, c):
            pltpu.async_copy(a_hbm.at[sidx.at[j]], buf.at[0], sem_a).wait()
            pltpu.async_copy(b_hbm.at[didx.at[j]], buf.at[0], sem_b,
                             add=True).wait()
            pltpu.sync_copy(
                buf.at[0],
                out_hbm.at[pl.ds(pl.multiple_of((TAIL0 + j) * CH, CH), CH)])
            return c

        lax.fori_loop(0, TAILC, tbody, 0)


_edge_call = pl.kernel(
    _edge_body,
    out_type=jax.ShapeDtypeStruct((E, 128), _F32),
    mesh=_MESH,
    scratch_types=[
        pltpu.VMEM((GSZ, CH), jnp.int32),
        pltpu.VMEM((GSZ, CH), jnp.int32),
        pltpu.VMEM((2, CH, 128), _F32),
        pltpu.SemaphoreType.DMA,
        pltpu.SemaphoreType.DMA,
        pltpu.SemaphoreType.DMA,
    ],
)


_PREC = lax.Precision.HIGHEST


def _mm_body(x_ref, w_ref, o_ref):
    o_ref[:] = jnp.dot(x_ref[:], w_ref[:], preferred_element_type=_F32,
                       precision=_PREC)


def _tc_matmul(x, w, rows_per_block=1280):
    m = x.shape[0]
    grid = m // rows_per_block
    return pl.pallas_call(
        _mm_body,
        grid=(grid,),
        in_specs=[
            pl.BlockSpec((rows_per_block, x.shape[1]), lambda i: (i, 0)),
            pl.BlockSpec(w.shape, lambda i: (0, 0)),
        ],
        out_specs=pl.BlockSpec((rows_per_block, w.shape[1]), lambda i: (i, 0)),
        out_shape=jax.ShapeDtypeStruct((m, w.shape[1]), _F32),
    )(x, w)


def _scale_body(h_ref, d_ref, g_ref, dinv_ref):
    dinv = lax.rsqrt(d_ref[0] + d_ref[1] + 1.0)
    dinv_ref[:] = dinv
    g_ref[:] = h_ref[:] * dinv


def _tc_scale(h, deg):
    grid = NPAD // 1280
    return pl.pallas_call(
        _scale_body,
        grid=(grid,),
        in_specs=[
            pl.BlockSpec((1280, 128), lambda i: (i, 0)),
            pl.BlockSpec((2, 1280, 1), lambda i: (0, i, 0)),
        ],
        out_specs=[
            pl.BlockSpec((1280, 128), lambda i: (i, 0)),
            pl.BlockSpec((1280, 1), lambda i: (i, 0)),
        ],
        out_shape=[
            jax.ShapeDtypeStruct((NPAD, 128), _F32),
            jax.ShapeDtypeStruct((NPAD, 1), _F32),
        ],
    )(h, deg)


def _layer_body(a_ref, g_ref, dinv_ref, b_ref, w_ref, o_ref):
    dinv = dinv_ref[:]
    h = (a_ref[0] + a_ref[1] + g_ref[:]) * dinv + b_ref[:]
    h = jnp.maximum(h, 0.0)
    o_ref[:] = jnp.dot(h, w_ref[:], preferred_element_type=_F32,
                       precision=_PREC) * dinv


def _tc_layer(agg, g, dinv, b, w):
    grid = NPAD // 1280
    return pl.pallas_call(
        _layer_body,
        grid=(grid,),
        in_specs=[
            pl.BlockSpec((2, 1280, 128), lambda i: (0, i, 0)),
            pl.BlockSpec((1280, 128), lambda i: (i, 0)),
            pl.BlockSpec((1280, 1), lambda i: (i, 0)),
            pl.BlockSpec((1, 128), lambda i: (0, 0)),
            pl.BlockSpec((128, 128), lambda i: (0, 0)),
        ],
        out_specs=pl.BlockSpec((1280, 128), lambda i: (i, 0)),
        out_shape=jax.ShapeDtypeStruct((NPAD, 128), _F32),
    )(agg, g, dinv, b, w)


def _final_node_body(a_ref, g_ref, dinv_ref, b_ref, wa_ref, wb_ref,
                     oa_ref, ob_ref):
    dinv = dinv_ref[:]
    h = (a_ref[0] + a_ref[1] + g_ref[:]) * dinv + b_ref[:]
    h = jnp.maximum(h, 0.0)
    oa_ref[:] = jnp.dot(h, wa_ref[:], preferred_element_type=_F32,
                        precision=_PREC)
    ob_ref[:] = jnp.dot(h, wb_ref[:], preferred_element_type=_F32,
                        precision=_PREC)


def _tc_final_node(agg, g, dinv, b, wa, wb):
    grid = NPAD // 1280
    return pl.pallas_call(
        _final_node_body,
        grid=(grid,),
        in_specs=[
            pl.BlockSpec((2, 1280, 128), lambda i: (0, i, 0)),
            pl.BlockSpec((1280, 128), lambda i: (i, 0)),
            pl.BlockSpec((1280, 1), lambda i: (i, 0)),
            pl.BlockSpec((1, 128), lambda i: (0, 0)),
            pl.BlockSpec((128, 128), lambda i: (0, 0)),
            pl.BlockSpec((128, 128), lambda i: (0, 0)),
        ],
        out_specs=[
            pl.BlockSpec((1280, 128), lambda i: (i, 0)),
            pl.BlockSpec((1280, 128), lambda i: (i, 0)),
        ],
        out_shape=[
            jax.ShapeDtypeStruct((NPAD, 128), _F32),
            jax.ShapeDtypeStruct((NPAD, 128), _F32),
        ],
    )(agg, g, dinv, b, wa, wb)


def _edge_mlp_body(s_ref, b1_ref, w2_ref, b2_ref, o_ref):
    z = jnp.maximum(s_ref[:] + b1_ref[:], 0.0)
    o_ref[:] = lax.dot_general(w2_ref[:], z, (((0,), (1,)), ((), ())),
                               preferred_element_type=_F32,
                               precision=_PREC) + b2_ref[:]


def _tc_edge_mlp(s, bm1, wm2, bm2):
    rows = 12800
    grid = E // rows
    return pl.pallas_call(
        _edge_mlp_body,
        grid=(grid,),
        in_specs=[
            pl.BlockSpec((rows, 128), lambda i: (i, 0)),
            pl.BlockSpec((1, 128), lambda i: (0, 0)),
            pl.BlockSpec((128, 16), lambda i: (0, 0)),
            pl.BlockSpec((16, 1), lambda i: (0, 0)),
        ],
        out_specs=pl.BlockSpec((16, rows), lambda i: (0, i)),
        out_shape=jax.ShapeDtypeStruct((16, E), _F32),
    )(s, bm1, wm2, bm2)


def kernel(x, edge_index, W1, b1, W2, b2, Wm1, bm1, Wm2, bm2):
    xp = jnp.pad(x, ((0, NPAD - N), (0, 0)))
    srcc = edge_index[0].reshape(C_CHUNKS, CH)
    dstc = edge_index[1].reshape(C_CHUNKS, CH)

    h1 = _tc_matmul(xp, W1)
    deg = _deg_call(dstc)
    g1, dinv = _tc_scale(h1, deg.reshape(NC, NPAD, 1))
    agg1 = _agg_call(g1, srcc, dstc)
    g2 = _tc_layer(agg1, g1, dinv, b1.reshape(1, 128), W2)
    agg2 = _agg_call(g2, srcc, dstc)
    A, B = _tc_final_node(agg2, g2, dinv, b2.reshape(1, 128),
                          Wm1[:128], Wm1[128:])
    S = _edge_call(A, B, srcc, dstc)
    pred_t = _tc_edge_mlp(S, bm1.reshape(1, 128), Wm2, bm2.reshape(16, 1))
    return pred_t.T


# speedup vs baseline: 17.2534x; 1.0755x over previous
import jax
import jax.numpy as jnp
from jax import lax
from jax.experimental import pallas as pl
from jax.experimental.pallas import tpu as pltpu
from jax.experimental.pallas import tpu_sc as plsc

N = 10000
E = 320000
NPAD = 10240
NC, NS, L = 2, 16, 16
CH = 128
C_CHUNKS = E // CH
CB = 2496 // 8
B0 = 156
TAIL0, TAILC = 2496, 4
GSZ = 40
WINC = 2456
RPT = NPAD // NS

_MESH = plsc.VectorSubcoreMesh(core_axis_name="c", subcore_axis_name="s",
                               num_cores=NC, num_subcores=NS)

_F32 = jnp.float32


def _chunk_range(cid, sid):
    base = jnp.where(cid == 0, 0, B0)
    wb = jnp.where(cid == 0, B0, CB - B0)
    t0 = 8 * (base + (sid * wb) // NS)
    t1 = 8 * (base + ((sid + 1) * wb) // NS)
    return t0, t1


def _is_tail_tile(cid, sid):
    return jnp.logical_and(cid == 1, sid == NS - 1)


def _group_geom(gi, t0, t1):
    gstart = t0 + gi * GSZ
    gcount = jnp.minimum(GSZ, t1 - gstart)
    win = pl.multiple_of(jnp.minimum(gstart, WINC), 8)
    roff = gstart - win
    return gstart, gcount, win, roff


def _ngroups(t0, t1):
    return (t1 - t0 + GSZ - 1) // GSZ


def _deg_body(dst_hbm, out_hbm, idx_v, ones_v, zb_v, acc_sh, sem):
    cid = lax.axis_index("c")
    sid = lax.axis_index("s")
    t0, t1 = _chunk_range(cid, sid)

    def fill_ones(i, c):
        ones_v[pl.ds(i * L, L)] = jnp.full((L,), 1.0, _F32)
        return c

    lax.fori_loop(0, CH // L, fill_ones, 0)

    def fill_zero(i, c):
        zb_v[pl.ds(i * L, L)] = jnp.zeros((L,), _F32)
        return c

    lax.fori_loop(0, RPT // L, fill_zero, 0)
    pltpu.sync_copy(zb_v, acc_sh.at[pl.ds(sid * RPT, RPT)])
    plsc.subcore_barrier()

    def group(gi, c):
        _, gcount, win, roff = _group_geom(gi, t0, t1)
        pltpu.async_copy(dst_hbm.at[pl.ds(win, GSZ)], idx_v, sem).wait()

        def body(j, c2):
            pltpu.sync_copy(ones_v, acc_sh.at[idx_v.at[roff + j]], add=True)
            return c2

        lax.fori_loop(0, gcount, body, 0)
        return c

    lax.fori_loop(0, _ngroups(t0, t1), group, 0)

    @pl.when(_is_tail_tile(cid, sid))
    def _():
        pltpu.async_copy(dst_hbm.at[pl.ds(TAIL0, TAILC)],
                         idx_v.at[pl.ds(0, TAILC)], sem).wait()

        def tbody(j, c):
            pltpu.sync_copy(ones_v, acc_sh.at[idx_v.at[j]], add=True)
            return c

        lax.fori_loop(0, TAILC, tbody, 0)

    plsc.subcore_barrier()
    pltpu.sync_copy(acc_sh.at[pl.ds(sid * RPT, RPT)],
                    out_hbm.at[cid, pl.ds(sid * RPT, RPT)])


_deg_call = pl.kernel(
    _deg_body,
    out_type=jax.ShapeDtypeStruct((NC, NPAD), _F32),
    mesh=_MESH,
    scratch_types=[
        pltpu.VMEM((GSZ, CH), jnp.int32),
        pltpu.VMEM((CH,), _F32),
        pltpu.VMEM((RPT,), _F32),
        pltpu.VMEM_SHARED((NPAD,), _F32),
        pltpu.SemaphoreType.DMA,
    ],
)


def _agg_body(g_hbm, src_hbm, dst_hbm, out_hbm, sidx, didx, buf, acc_sh, sem):
    cid = lax.axis_index("c")
    sid = lax.axis_index("s")
    t0, t1 = _chunk_range(cid, sid)

    def fill_zero(i, c):
        r = i // 8
        col = i % 8
        buf[0, r, pl.ds(col * L, L)] = jnp.zeros((L,), _F32)
        return c

    lax.fori_loop(0, CH * 8, fill_zero, 0)
    for t in range(RPT // CH):
        pltpu.sync_copy(buf.at[0], acc_sh.at[pl.ds(sid * RPT + t * CH, CH)])
    plsc.subcore_barrier()

    def group(gi, c):
        _, gcount, win, roff = _group_geom(gi, t0, t1)
        pltpu.async_copy(src_hbm.at[pl.ds(win, GSZ)], sidx, sem).wait()
        pltpu.async_copy(dst_hbm.at[pl.ds(win, GSZ)], didx, sem).wait()

        pltpu.async_copy(g_hbm.at[sidx.at[roff]], buf.at[0], sem)

        def body(j, c2):
            b = j % 2
            pltpu.make_async_copy(g_hbm.at[sidx.at[roff + j]], buf.at[b],
                                  sem).wait()

            @pl.when(j + 1 < gcount)
            def _():
                pltpu.async_copy(g_hbm.at[sidx.at[roff + j + 1]],
                                 buf.at[(j + 1) % 2], sem)

            pltpu.sync_copy(buf.at[b], acc_sh.at[didx.at[roff + j]], add=True)
            return c2

        lax.fori_loop(0, gcount, body, 0)
        return c

    lax.fori_loop(0, _ngroups(t0, t1), group, 0)

    @pl.when(_is_tail_tile(cid, sid))
    def _():
        pltpu.async_copy(src_hbm.at[pl.ds(TAIL0, TAILC)],
                         sidx.at[pl.ds(0, TAILC)], sem).wait()
        pltpu.async_copy(dst_hbm.at[pl.ds(TAIL0, TAILC)],
                         didx.at[pl.ds(0, TAILC)], sem).wait()

        def tbody(j, c):
            pltpu.async_copy(g_hbm.at[sidx.at[j]], buf.at[0], sem).wait()
            pltpu.sync_copy(buf.at[0], acc_sh.at[didx.at[j]], add=True)
            return c

        lax.fori_loop(0, TAILC, tbody, 0)

    plsc.subcore_barrier()
    pltpu.sync_copy(acc_sh.at[pl.ds(sid * RPT, RPT)],
                    out_hbm.at[cid, pl.ds(sid * RPT, RPT)])


_agg_call = pl.kernel(
    _agg_body,
    out_type=jax.ShapeDtypeStruct((NC, NPAD, 128), _F32),
    mesh=_MESH,
    scratch_types=[
        pltpu.VMEM((GSZ, CH), jnp.int32),
        pltpu.VMEM((GSZ, CH), jnp.int32),
        pltpu.VMEM((2, CH, 128), _F32),
        pltpu.VMEM_SHARED((NPAD, 128), _F32),
        pltpu.SemaphoreType.DMA,
    ],
)


HSPLIT_B = 160
E_H0 = HSPLIT_B * 8 * CH
E_H1 = E - E_H0


def _make_edge_half(clo_b, nblocks, has_tail, e_lo, e_cnt):
    wb = nblocks // 2

    def _edge_body(a_hbm, b_hbm, src_hbm, dst_hbm, out_hbm, sidx, didx, buf,
                   sem_a, sem_b, sem_o):
        cid = lax.axis_index("c")
        sid = lax.axis_index("s")
        base_b = jnp.where(cid == 0, 0, wb)
        t0 = 8 * (clo_b + base_b + (sid * wb) // NS)
        t1 = 8 * (clo_b + base_b + ((sid + 1) * wb) // NS)

        def group(gi, c):
            gstart, gcount, win, roff = _group_geom(gi, t0, t1)
            pltpu.async_copy(src_hbm.at[pl.ds(win, GSZ)], sidx, sem_a).wait()
            pltpu.async_copy(dst_hbm.at[pl.ds(win, GSZ)], didx, sem_a).wait()

            pltpu.async_copy(a_hbm.at[sidx.at[roff]], buf.at[0], sem_a)

            def body(j, c2):
                b = j % 2
                pltpu.make_async_copy(a_hbm.at[sidx.at[roff + j]], buf.at[b],
                                      sem_a).wait()
                pltpu.async_copy(b_hbm.at[didx.at[roff + j]], buf.at[b],
                                 sem_b, add=True)

                @pl.when(j >= 1)
                def _():
                    pltpu.make_async_copy(
                        buf.at[1 - b],
                        out_hbm.at[pl.ds(
                            pl.multiple_of((gstart + j - 1) * CH - e_lo, CH),
                            CH)],
                        sem_o).wait()

                @pl.when(j + 1 < gcount)
                def _():
                    pltpu.async_copy(a_hbm.at[sidx.at[roff + j + 1]],
                                     buf.at[1 - b], sem_a)

                pltpu.make_async_copy(b_hbm.at[didx.at[roff + j]], buf.at[b],
                                      sem_b).wait()
                pltpu.async_copy(
                    buf.at[b],
                    out_hbm.at[pl.ds(
                        pl.multiple_of((gstart + j) * CH - e_lo, CH), CH)],
                    sem_o)
                return c2

            lax.fori_loop(0, gcount, body, 0)
            pltpu.make_async_copy(
                buf.at[(gcount - 1) % 2],
                out_hbm.at[pl.ds(
                    pl.multiple_of((gstart + gcount - 1) * CH - e_lo, CH),
                    CH)],
                sem_o).wait()
            return c

        lax.fori_loop(0, (t1 - t0 + GSZ - 1) // GSZ, group, 0)

        if has_tail:
            @pl.when(_is_tail_tile(cid, sid))
            def _():
                pltpu.async_copy(src_hbm.at[pl.ds(TAIL0, TAILC)],
                                 sidx.at[pl.ds(0, TAILC)], sem_a).wait()
                pltpu.async_copy(dst_hbm.at[pl.ds(TAIL0, TAILC)],
                                 didx.at[pl.ds(0, TAILC)], sem_a).wait()

                def tbody(j, c):
                    pltpu.async_copy(a_hbm.at[sidx.at[j]], buf.at[0],
                                     sem_a).wait()
                    pltpu.async_copy(b_hbm.at[didx.at[j]], buf.at[0], sem_b,
                                     add=True).wait()
                    pltpu.sync_copy(
                        buf.at[0],
                        out_hbm.at[pl.ds(
                            pl.multiple_of((TAIL0 + j) * CH - e_lo, CH), CH)])
                    return c

                lax.fori_loop(0, TAILC, tbody, 0)

    return pl.kernel(
        _edge_body,
        out_type=jax.ShapeDtypeStruct((e_cnt, 128), _F32),
        mesh=_MESH,
        scratch_types=[
            pltpu.VMEM((GSZ, CH), jnp.int32),
            pltpu.VMEM((GSZ, CH), jnp.int32),
            pltpu.VMEM((2, CH, 128), _F32),
            pltpu.SemaphoreType.DMA,
            pltpu.SemaphoreType.DMA,
            pltpu.SemaphoreType.DMA,
        ],
    )


_edge_call_h0 = _make_edge_half(0, HSPLIT_B, False, 0, E_H0)
_edge_call_h1 = _make_edge_half(HSPLIT_B, CB - HSPLIT_B, True, E_H0, E_H1)


_PREC = lax.Precision.HIGHEST


def _mm_body(x_ref, w_ref, o_ref):
    o_ref[:] = jnp.dot(x_ref[:], w_ref[:], preferred_element_type=_F32,
                       precision=_PREC)


def _tc_matmul(x, w, rows_per_block=1280):
    m = x.shape[0]
    grid = m // rows_per_block
    return pl.pallas_call(
        _mm_body,
        grid=(grid,),
        in_specs=[
            pl.BlockSpec((rows_per_block, x.shape[1]), lambda i: (i, 0)),
            pl.BlockSpec(w.shape, lambda i: (0, 0)),
        ],
        out_specs=pl.BlockSpec((rows_per_block, w.shape[1]), lambda i: (i, 0)),
        out_shape=jax.ShapeDtypeStruct((m, w.shape[1]), _F32),
    )(x, w)


def _scale_body(h_ref, d_ref, g_ref, dinv_ref):
    dinv = lax.rsqrt(d_ref[0] + d_ref[1] + 1.0)
    dinv_ref[:] = dinv
    g_ref[:] = h_ref[:] * dinv


def _tc_scale(h, deg):
    grid = NPAD // 1280
    return pl.pallas_call(
        _scale_body,
        grid=(grid,),
        in_specs=[
            pl.BlockSpec((1280, 128), lambda i: (i, 0)),
            pl.BlockSpec((2, 1280, 1), lambda i: (0, i, 0)),
        ],
        out_specs=[
            pl.BlockSpec((1280, 128), lambda i: (i, 0)),
            pl.BlockSpec((1280, 1), lambda i: (i, 0)),
        ],
        out_shape=[
            jax.ShapeDtypeStruct((NPAD, 128), _F32),
            jax.ShapeDtypeStruct((NPAD, 1), _F32),
        ],
    )(h, deg)


def _layer_body(a_ref, g_ref, dinv_ref, b_ref, w_ref, o_ref):
    dinv = dinv_ref[:]
    h = (a_ref[0] + a_ref[1] + g_ref[:]) * dinv + b_ref[:]
    h = jnp.maximum(h, 0.0)
    o_ref[:] = jnp.dot(h, w_ref[:], preferred_element_type=_F32,
                       precision=_PREC) * dinv


def _tc_layer(agg, g, dinv, b, w):
    grid = NPAD // 1280
    return pl.pallas_call(
        _layer_body,
        grid=(grid,),
        in_specs=[
            pl.BlockSpec((2, 1280, 128), lambda i: (0, i, 0)),
            pl.BlockSpec((1280, 128), lambda i: (i, 0)),
            pl.BlockSpec((1280, 1), lambda i: (i, 0)),
            pl.BlockSpec((1, 128), lambda i: (0, 0)),
            pl.BlockSpec((128, 128), lambda i: (0, 0)),
        ],
        out_specs=pl.BlockSpec((1280, 128), lambda i: (i, 0)),
        out_shape=jax.ShapeDtypeStruct((NPAD, 128), _F32),
    )(agg, g, dinv, b, w)


def _final_node_body(a_ref, g_ref, dinv_ref, b_ref, wa_ref, wb_ref,
                     oa_ref, ob_ref):
    dinv = dinv_ref[:]
    h = (a_ref[0] + a_ref[1] + g_ref[:]) * dinv + b_ref[:]
    h = jnp.maximum(h, 0.0)
    oa_ref[:] = jnp.dot(h, wa_ref[:], preferred_element_type=_F32,
                        precision=_PREC)
    ob_ref[:] = jnp.dot(h, wb_ref[:], preferred_element_type=_F32,
                        precision=_PREC)


def _tc_final_node(agg, g, dinv, b, wa, wb):
    grid = NPAD // 1280
    return pl.pallas_call(
        _final_node_body,
        grid=(grid,),
        in_specs=[
            pl.BlockSpec((2, 1280, 128), lambda i: (0, i, 0)),
            pl.BlockSpec((1280, 128), lambda i: (i, 0)),
            pl.BlockSpec((1280, 1), lambda i: (i, 0)),
            pl.BlockSpec((1, 128), lambda i: (0, 0)),
            pl.BlockSpec((128, 128), lambda i: (0, 0)),
            pl.BlockSpec((128, 128), lambda i: (0, 0)),
        ],
        out_specs=[
            pl.BlockSpec((1280, 128), lambda i: (i, 0)),
            pl.BlockSpec((1280, 128), lambda i: (i, 0)),
        ],
        out_shape=[
            jax.ShapeDtypeStruct((NPAD, 128), _F32),
            jax.ShapeDtypeStruct((NPAD, 128), _F32),
        ],
    )(agg, g, dinv, b, wa, wb)


def _edge_mlp_body(s_ref, b1_ref, w2_ref, b2_ref, o_ref):
    z = jnp.maximum(s_ref[:] + b1_ref[:], 0.0).astype(jnp.bfloat16)
    w2 = w2_ref[:].astype(jnp.bfloat16)
    o_ref[:] = lax.dot_general(w2, z, (((0,), (1,)), ((), ())),
                               preferred_element_type=_F32) + b2_ref[:]


def _edge_mlp_body2(pred_ref, s_ref, b1_ref, w2_ref, b2_ref, o_ref):
    del pred_ref
    _edge_mlp_body(s_ref, b1_ref, w2_ref, b2_ref, o_ref)


def _tc_edge_mlp_h0(s, bm1, wm2, bm2):
    rows = 16384
    grid = E_H0 // rows
    return pl.pallas_call(
        _edge_mlp_body,
        grid=(grid,),
        in_specs=[
            pl.BlockSpec((rows, 128), lambda i: (i, 0)),
            pl.BlockSpec((1, 128), lambda i: (0, 0)),
            pl.BlockSpec((128, 16), lambda i: (0, 0)),
            pl.BlockSpec((16, 1), lambda i: (0, 0)),
        ],
        out_specs=pl.BlockSpec((16, rows), lambda i: (0, i)),
        out_shape=jax.ShapeDtypeStruct((16, E), _F32),
    )(s, bm1, wm2, bm2)


def _tc_edge_mlp_h1(pred, s, bm1, wm2, bm2):
    rows = 2560
    grid = E_H1 // rows
    off = E_H0 // rows
    return pl.pallas_call(
        _edge_mlp_body2,
        grid=(grid,),
        in_specs=[
            pl.BlockSpec(memory_space=pl.ANY),
            pl.BlockSpec((rows, 128), lambda i: (i, 0)),
            pl.BlockSpec((1, 128), lambda i: (0, 0)),
            pl.BlockSpec((128, 16), lambda i: (0, 0)),
            pl.BlockSpec((16, 1), lambda i: (0, 0)),
        ],
        out_specs=pl.BlockSpec((16, rows), lambda i: (0, off + i)),
        out_shape=jax.ShapeDtypeStruct((16, E), _F32),
        input_output_aliases={0: 0},
    )(pred, s, bm1, wm2, bm2)


def kernel(x, edge_index, W1, b1, W2, b2, Wm1, bm1, Wm2, bm2):
    xp = jnp.pad(x, ((0, NPAD - N), (0, 0)))
    srcc = edge_index[0].reshape(C_CHUNKS, CH)
    dstc = edge_index[1].reshape(C_CHUNKS, CH)

    h1 = _tc_matmul(xp, W1)
    deg = _deg_call(dstc)
    g1, dinv = _tc_scale(h1, deg.reshape(NC, NPAD, 1))
    agg1 = _agg_call(g1, srcc, dstc)
    g2 = _tc_layer(agg1, g1, dinv, b1.reshape(1, 128), W2)
    agg2 = _agg_call(g2, srcc, dstc)
    A, B = _tc_final_node(agg2, g2, dinv, b2.reshape(1, 128),
                          Wm1[:128], Wm1[128:])
    s0 = _edge_call_h0(A, B, srcc, dstc)
    s1 = _edge_call_h1(A, B, srcc, dstc)
    pred_t = _tc_edge_mlp_h0(s0, bm1.reshape(1, 128), Wm2, bm2.reshape(16, 1))
    pred_t = _tc_edge_mlp_h1(pred_t, s1, bm1.reshape(1, 128), Wm2,
                             bm2.reshape(16, 1))
    return pred_t.T


# speedup vs baseline: 17.9960x; 1.0430x over previous
import jax
import jax.numpy as jnp
from jax import lax
from jax.experimental import pallas as pl
from jax.experimental.pallas import tpu as pltpu
from jax.experimental.pallas import tpu_sc as plsc

N = 10000
E = 320000
NPAD = 10240
NC, NS, L = 2, 16, 16
CH = 128
C_CHUNKS = E // CH
CB = 2496 // 8
B0 = 156
TAIL0, TAILC = 2496, 4
GSZ = 40
WINC = 2456
RPT = NPAD // NS

_MESH = plsc.VectorSubcoreMesh(core_axis_name="c", subcore_axis_name="s",
                               num_cores=NC, num_subcores=NS)

_F32 = jnp.float32


def _chunk_range(cid, sid):
    base = jnp.where(cid == 0, 0, B0)
    wb = jnp.where(cid == 0, B0, CB - B0)
    t0 = 8 * (base + (sid * wb) // NS)
    t1 = 8 * (base + ((sid + 1) * wb) // NS)
    return t0, t1


def _is_tail_tile(cid, sid):
    return jnp.logical_and(cid == 1, sid == NS - 1)


def _group_geom(gi, t0, t1):
    gstart = t0 + gi * GSZ
    gcount = jnp.minimum(GSZ, t1 - gstart)
    win = pl.multiple_of(jnp.minimum(gstart, WINC), 8)
    roff = gstart - win
    return gstart, gcount, win, roff


def _ngroups(t0, t1):
    return (t1 - t0 + GSZ - 1) // GSZ


def _deg_body(dst_hbm, out_hbm, idx_v, ones_v, zb_v, acc_sh, sem):
    cid = lax.axis_index("c")
    sid = lax.axis_index("s")
    t0, t1 = _chunk_range(cid, sid)

    def fill_ones(i, c):
        ones_v[pl.ds(i * L, L)] = jnp.full((L,), 1.0, _F32)
        return c

    lax.fori_loop(0, CH // L, fill_ones, 0)

    def fill_zero(i, c):
        zb_v[pl.ds(i * L, L)] = jnp.zeros((L,), _F32)
        return c

    lax.fori_loop(0, RPT // L, fill_zero, 0)
    pltpu.sync_copy(zb_v, acc_sh.at[pl.ds(sid * RPT, RPT)])
    plsc.subcore_barrier()

    def group(gi, c):
        _, gcount, win, roff = _group_geom(gi, t0, t1)
        pltpu.async_copy(dst_hbm.at[pl.ds(win, GSZ)], idx_v, sem).wait()

        def body(j, c2):
            pltpu.sync_copy(ones_v, acc_sh.at[idx_v.at[roff + j]], add=True)
            return c2

        lax.fori_loop(0, gcount, body, 0)
        return c

    lax.fori_loop(0, _ngroups(t0, t1), group, 0)

    @pl.when(_is_tail_tile(cid, sid))
    def _():
        pltpu.async_copy(dst_hbm.at[pl.ds(TAIL0, TAILC)],
                         idx_v.at[pl.ds(0, TAILC)], sem).wait()

        def tbody(j, c):
            pltpu.sync_copy(ones_v, acc_sh.at[idx_v.at[j]], add=True)
            return c

        lax.fori_loop(0, TAILC, tbody, 0)

    plsc.subcore_barrier()
    pltpu.sync_copy(acc_sh.at[pl.ds(sid * RPT, RPT)],
                    out_hbm.at[cid, pl.ds(sid * RPT, RPT)])


_deg_call = pl.kernel(
    _deg_body,
    out_type=jax.ShapeDtypeStruct((NC, NPAD), _F32),
    mesh=_MESH,
    scratch_types=[
        pltpu.VMEM((GSZ, CH), jnp.int32),
        pltpu.VMEM((CH,), _F32),
        pltpu.VMEM((RPT,), _F32),
        pltpu.VMEM_SHARED((NPAD,), _F32),
        pltpu.SemaphoreType.DMA,
    ],
)


def _agg_body(g_hbm, src_hbm, dst_hbm, out_hbm, sidx, didx, buf, acc_sh, sem):
    cid = lax.axis_index("c")
    sid = lax.axis_index("s")
    t0, t1 = _chunk_range(cid, sid)

    def fill_zero(i, c):
        r = i // 8
        col = i % 8
        buf[0, r, pl.ds(col * L, L)] = jnp.zeros((L,), _F32)
        return c

    lax.fori_loop(0, CH * 8, fill_zero, 0)
    for t in range(RPT // CH):
        pltpu.sync_copy(buf.at[0], acc_sh.at[pl.ds(sid * RPT + t * CH, CH)])
    plsc.subcore_barrier()

    def group(gi, c):
        _, gcount, win, roff = _group_geom(gi, t0, t1)
        pltpu.async_copy(src_hbm.at[pl.ds(win, GSZ)], sidx, sem).wait()
        pltpu.async_copy(dst_hbm.at[pl.ds(win, GSZ)], didx, sem).wait()

        pltpu.async_copy(g_hbm.at[sidx.at[roff]], buf.at[0], sem)

        def body(j, c2):
            b = j % 2
            pltpu.make_async_copy(g_hbm.at[sidx.at[roff + j]], buf.at[b],
                                  sem).wait()

            @pl.when(j + 1 < gcount)
            def _():
                pltpu.async_copy(g_hbm.at[sidx.at[roff + j + 1]],
                                 buf.at[(j + 1) % 2], sem)

            pltpu.sync_copy(buf.at[b], acc_sh.at[didx.at[roff + j]], add=True)
            return c2

        lax.fori_loop(0, gcount, body, 0)
        return c

    lax.fori_loop(0, _ngroups(t0, t1), group, 0)

    @pl.when(_is_tail_tile(cid, sid))
    def _():
        pltpu.async_copy(src_hbm.at[pl.ds(TAIL0, TAILC)],
                         sidx.at[pl.ds(0, TAILC)], sem).wait()
        pltpu.async_copy(dst_hbm.at[pl.ds(TAIL0, TAILC)],
                         didx.at[pl.ds(0, TAILC)], sem).wait()

        def tbody(j, c):
            pltpu.async_copy(g_hbm.at[sidx.at[j]], buf.at[0], sem).wait()
            pltpu.sync_copy(buf.at[0], acc_sh.at[didx.at[j]], add=True)
            return c

        lax.fori_loop(0, TAILC, tbody, 0)

    plsc.subcore_barrier()
    pltpu.sync_copy(acc_sh.at[pl.ds(sid * RPT, RPT)],
                    out_hbm.at[cid, pl.ds(sid * RPT, RPT)])


_agg_call = pl.kernel(
    _agg_body,
    out_type=jax.ShapeDtypeStruct((NC, NPAD, 128), _F32),
    mesh=_MESH,
    scratch_types=[
        pltpu.VMEM((GSZ, CH), jnp.int32),
        pltpu.VMEM((GSZ, CH), jnp.int32),
        pltpu.VMEM((2, CH, 128), _F32),
        pltpu.VMEM_SHARED((NPAD, 128), _F32),
        pltpu.SemaphoreType.DMA,
    ],
)


def _edge_body(a_hbm, b_hbm, src_hbm, dst_hbm, out_hbm, sidx, didx, buf,
               sem_a, sem_b, sem_o):
    cid = lax.axis_index("c")
    sid = lax.axis_index("s")
    t0, t1 = _chunk_range(cid, sid)

    def group(gi, c):
        gstart, gcount, win, roff = _group_geom(gi, t0, t1)
        pltpu.async_copy(src_hbm.at[pl.ds(win, GSZ)], sidx, sem_a).wait()
        pltpu.async_copy(dst_hbm.at[pl.ds(win, GSZ)], didx, sem_a).wait()

        pltpu.async_copy(a_hbm.at[sidx.at[roff]], buf.at[0], sem_a)

        def body(j, c2):
            b = j % 2
            pltpu.make_async_copy(a_hbm.at[sidx.at[roff + j]], buf.at[b],
                                  sem_a).wait()
            pltpu.async_copy(b_hbm.at[didx.at[roff + j]], buf.at[b], sem_b,
                             add=True)

            @pl.when(j >= 1)
            def _():
                pltpu.make_async_copy(
                    buf.at[1 - b],
                    out_hbm.at[pl.ds(pl.multiple_of((gstart + j - 1) * CH, CH),
                                     CH)],
                    sem_o).wait()

            @pl.when(j + 1 < gcount)
            def _():
                pltpu.async_copy(a_hbm.at[sidx.at[roff + j + 1]],
                                 buf.at[1 - b], sem_a)

            pltpu.make_async_copy(b_hbm.at[didx.at[roff + j]], buf.at[b],
                                  sem_b).wait()
            pltpu.async_copy(
                buf.at[b],
                out_hbm.at[pl.ds(pl.multiple_of((gstart + j) * CH, CH), CH)],
                sem_o)
            return c2

        lax.fori_loop(0, gcount, body, 0)
        pltpu.make_async_copy(
            buf.at[(gcount - 1) % 2],
            out_hbm.at[pl.ds(pl.multiple_of((gstart + gcount - 1) * CH, CH),
                             CH)],
            sem_o).wait()
        return c

    lax.fori_loop(0, _ngroups(t0, t1), group, 0)

    @pl.when(_is_tail_tile(cid, sid))
    def _():
        pltpu.async_copy(src_hbm.at[pl.ds(TAIL0, TAILC)],
                         sidx.at[pl.ds(0, TAILC)], sem_a).wait()
        pltpu.async_copy(dst_hbm.at[pl.ds(TAIL0, TAILC)],
                         didx.at[pl.ds(0, TAILC)], sem_a).wait()

        def tbody(j, c):
            pltpu.async_copy(a_hbm.at[sidx.at[j]], buf.at[0], sem_a).wait()
            pltpu.async_copy(b_hbm.at[didx.at[j]], buf.at[0], sem_b,
                             add=True).wait()
            pltpu.sync_copy(
                buf.at[0],
                out_hbm.at[pl.ds(pl.multiple_of((TAIL0 + j) * CH, CH), CH)])
            return c

        lax.fori_loop(0, TAILC, tbody, 0)


_edge_call = pl.kernel(
    _edge_body,
    out_type=jax.ShapeDtypeStruct((E, 128), _F32),
    mesh=_MESH,
    scratch_types=[
        pltpu.VMEM((GSZ, CH), jnp.int32),
        pltpu.VMEM((GSZ, CH), jnp.int32),
        pltpu.VMEM((2, CH, 128), _F32),
        pltpu.SemaphoreType.DMA,
        pltpu.SemaphoreType.DMA,
        pltpu.SemaphoreType.DMA,
    ],
)


_PREC = lax.Precision.HIGHEST


def _mm_body(x_ref, w_ref, o_ref):
    o_ref[:] = jnp.dot(x_ref[:], w_ref[:], preferred_element_type=_F32,
                       precision=_PREC)


def _tc_matmul(x, w, rows_per_block=1280):
    m = x.shape[0]
    grid = m // rows_per_block
    return pl.pallas_call(
        _mm_body,
        grid=(grid,),
        in_specs=[
            pl.BlockSpec((rows_per_block, x.shape[1]), lambda i: (i, 0)),
            pl.BlockSpec(w.shape, lambda i: (0, 0)),
        ],
        out_specs=pl.BlockSpec((rows_per_block, w.shape[1]), lambda i: (i, 0)),
        out_shape=jax.ShapeDtypeStruct((m, w.shape[1]), _F32),
    )(x, w)


def _scale_body(h_ref, d_ref, g_ref, dinv_ref):
    dinv = lax.rsqrt(d_ref[0] + d_ref[1] + 1.0)
    dinv_ref[:] = dinv
    g_ref[:] = h_ref[:] * dinv


def _tc_scale(h, deg):
    grid = NPAD // 1280
    return pl.pallas_call(
        _scale_body,
        grid=(grid,),
        in_specs=[
            pl.BlockSpec((1280, 128), lambda i: (i, 0)),
            pl.BlockSpec((2, 1280, 1), lambda i: (0, i, 0)),
        ],
        out_specs=[
            pl.BlockSpec((1280, 128), lambda i: (i, 0)),
            pl.BlockSpec((1280, 1), lambda i: (i, 0)),
        ],
        out_shape=[
            jax.ShapeDtypeStruct((NPAD, 128), _F32),
            jax.ShapeDtypeStruct((NPAD, 1), _F32),
        ],
    )(h, deg)


def _layer_body(a_ref, g_ref, dinv_ref, b_ref, w_ref, o_ref):
    dinv = dinv_ref[:]
    h = (a_ref[0] + a_ref[1] + g_ref[:]) * dinv + b_ref[:]
    h = jnp.maximum(h, 0.0)
    o_ref[:] = jnp.dot(h, w_ref[:], preferred_element_type=_F32,
                       precision=_PREC) * dinv


def _tc_layer(agg, g, dinv, b, w):
    grid = NPAD // 1280
    return pl.pallas_call(
        _layer_body,
        grid=(grid,),
        in_specs=[
            pl.BlockSpec((2, 1280, 128), lambda i: (0, i, 0)),
            pl.BlockSpec((1280, 128), lambda i: (i, 0)),
            pl.BlockSpec((1280, 1), lambda i: (i, 0)),
            pl.BlockSpec((1, 128), lambda i: (0, 0)),
            pl.BlockSpec((128, 128), lambda i: (0, 0)),
        ],
        out_specs=pl.BlockSpec((1280, 128), lambda i: (i, 0)),
        out_shape=jax.ShapeDtypeStruct((NPAD, 128), _F32),
    )(agg, g, dinv, b, w)


def _final_node_body(a_ref, g_ref, dinv_ref, b_ref, wa_ref, wb_ref,
                     oa_ref, ob_ref):
    dinv = dinv_ref[:]
    h = (a_ref[0] + a_ref[1] + g_ref[:]) * dinv + b_ref[:]
    h = jnp.maximum(h, 0.0)
    oa_ref[:] = jnp.dot(h, wa_ref[:], preferred_element_type=_F32,
                        precision=_PREC)
    ob_ref[:] = jnp.dot(h, wb_ref[:], preferred_element_type=_F32,
                        precision=_PREC)


def _tc_final_node(agg, g, dinv, b, wa, wb):
    grid = NPAD // 1280
    return pl.pallas_call(
        _final_node_body,
        grid=(grid,),
        in_specs=[
            pl.BlockSpec((2, 1280, 128), lambda i: (0, i, 0)),
            pl.BlockSpec((1280, 128), lambda i: (i, 0)),
            pl.BlockSpec((1280, 1), lambda i: (i, 0)),
            pl.BlockSpec((1, 128), lambda i: (0, 0)),
            pl.BlockSpec((128, 128), lambda i: (0, 0)),
            pl.BlockSpec((128, 128), lambda i: (0, 0)),
        ],
        out_specs=[
            pl.BlockSpec((1280, 128), lambda i: (i, 0)),
            pl.BlockSpec((1280, 128), lambda i: (i, 0)),
        ],
        out_shape=[
            jax.ShapeDtypeStruct((NPAD, 128), _F32),
            jax.ShapeDtypeStruct((NPAD, 128), _F32),
        ],
    )(agg, g, dinv, b, wa, wb)


def _edge_mlp_body(s_ref, b1_ref, w2_ref, b2_ref, o_ref):
    z = jnp.maximum(s_ref[:] + b1_ref[:], 0.0).astype(jnp.bfloat16)
    w2 = w2_ref[:].astype(jnp.bfloat16)
    o_ref[:] = lax.dot_general(w2, z, (((0,), (1,)), ((), ())),
                               preferred_element_type=_F32) + b2_ref[:]


def _tc_edge_mlp(s, bm1, wm2, bm2):
    rows = 12800
    grid = E // rows
    return pl.pallas_call(
        _edge_mlp_body,
        grid=(grid,),
        in_specs=[
            pl.BlockSpec((rows, 128), lambda i: (i, 0)),
            pl.BlockSpec((1, 128), lambda i: (0, 0)),
            pl.BlockSpec((128, 16), lambda i: (0, 0)),
            pl.BlockSpec((16, 1), lambda i: (0, 0)),
        ],
        out_specs=pl.BlockSpec((16, rows), lambda i: (0, i)),
        out_shape=jax.ShapeDtypeStruct((16, E), _F32),
    )(s, bm1, wm2, bm2)


def kernel(x, edge_index, W1, b1, W2, b2, Wm1, bm1, Wm2, bm2):
    xp = jnp.pad(x, ((0, NPAD - N), (0, 0)))
    srcc = edge_index[0].reshape(C_CHUNKS, CH)
    dstc = edge_index[1].reshape(C_CHUNKS, CH)

    h1 = _tc_matmul(xp, W1)
    deg = _deg_call(dstc)
    g1, dinv = _tc_scale(h1, deg.reshape(NC, NPAD, 1))
    agg1 = _agg_call(g1, srcc, dstc)
    g2 = _tc_layer(agg1, g1, dinv, b1.reshape(1, 128), W2)
    agg2 = _agg_call(g2, srcc, dstc)
    A, B = _tc_final_node(agg2, g2, dinv, b2.reshape(1, 128),
                          Wm1[:128], Wm1[128:])
    S = _edge_call(A, B, srcc, dstc)
    pred_t = _tc_edge_mlp(S, bm1.reshape(1, 128), Wm2, bm2.reshape(16, 1))
    return pred_t.T


# speedup vs baseline: 18.0467x; 1.0028x over previous
import jax
import jax.numpy as jnp
from jax import lax
from jax.experimental import pallas as pl
from jax.experimental.pallas import tpu as pltpu
from jax.experimental.pallas import tpu_sc as plsc

N = 10000
E = 320000
NPAD = 10240
NC, NS, L = 2, 16, 16
CH = 128
C_CHUNKS = E // CH
CB = 2496 // 8
B0 = 156
TAIL0, TAILC = 2496, 4
GSZ = 40
WINC = 2456
RPT = NPAD // NS

_MESH = plsc.VectorSubcoreMesh(core_axis_name="c", subcore_axis_name="s",
                               num_cores=NC, num_subcores=NS)

_F32 = jnp.float32


def _chunk_range(cid, sid):
    base = jnp.where(cid == 0, 0, B0)
    wb = jnp.where(cid == 0, B0, CB - B0)
    t0 = 8 * (base + (sid * wb) // NS)
    t1 = 8 * (base + ((sid + 1) * wb) // NS)
    return t0, t1


def _is_tail_tile(cid, sid):
    return jnp.logical_and(cid == 1, sid == NS - 1)


def _group_geom(gi, t0, t1):
    gstart = t0 + gi * GSZ
    gcount = jnp.minimum(GSZ, t1 - gstart)
    win = pl.multiple_of(jnp.minimum(gstart, WINC), 8)
    roff = gstart - win
    return gstart, gcount, win, roff


def _ngroups(t0, t1):
    return (t1 - t0 + GSZ - 1) // GSZ


def _deg_body(dst_hbm, out_hbm, idx_v, ones_v, zb_v, acc_sh, sem):
    cid = lax.axis_index("c")
    sid = lax.axis_index("s")
    t0, t1 = _chunk_range(cid, sid)

    def fill_ones(i, c):
        ones_v[pl.ds(i * L, L)] = jnp.full((L,), 1.0, _F32)
        return c

    lax.fori_loop(0, CH // L, fill_ones, 0)

    def fill_zero(i, c):
        zb_v[pl.ds(i * L, L)] = jnp.zeros((L,), _F32)
        return c

    lax.fori_loop(0, RPT // L, fill_zero, 0)
    pltpu.sync_copy(zb_v, acc_sh.at[pl.ds(sid * RPT, RPT)])
    plsc.subcore_barrier()

    def group(gi, c):
        _, gcount, win, roff = _group_geom(gi, t0, t1)
        pltpu.async_copy(dst_hbm.at[pl.ds(win, GSZ)], idx_v, sem).wait()

        def body(j, c2):
            pltpu.sync_copy(ones_v, acc_sh.at[idx_v.at[roff + j]], add=True)
            return c2

        lax.fori_loop(0, gcount, body, 0)
        return c

    lax.fori_loop(0, _ngroups(t0, t1), group, 0)

    @pl.when(_is_tail_tile(cid, sid))
    def _():
        pltpu.async_copy(dst_hbm.at[pl.ds(TAIL0, TAILC)],
                         idx_v.at[pl.ds(0, TAILC)], sem).wait()

        def tbody(j, c):
            pltpu.sync_copy(ones_v, acc_sh.at[idx_v.at[j]], add=True)
            return c

        lax.fori_loop(0, TAILC, tbody, 0)

    plsc.subcore_barrier()
    pltpu.sync_copy(acc_sh.at[pl.ds(sid * RPT, RPT)],
                    out_hbm.at[cid, pl.ds(sid * RPT, RPT)])


_deg_call = pl.kernel(
    _deg_body,
    out_type=jax.ShapeDtypeStruct((NC, NPAD), _F32),
    mesh=_MESH,
    scratch_types=[
        pltpu.VMEM((GSZ, CH), jnp.int32),
        pltpu.VMEM((CH,), _F32),
        pltpu.VMEM((RPT,), _F32),
        pltpu.VMEM_SHARED((NPAD,), _F32),
        pltpu.SemaphoreType.DMA,
    ],
)


def _agg_body(g_hbm, src_hbm, dst_hbm, out_hbm, sidx, didx, buf, acc_sh, sem):
    cid = lax.axis_index("c")
    sid = lax.axis_index("s")
    t0, t1 = _chunk_range(cid, sid)

    def fill_zero(i, c):
        r = i // 8
        col = i % 8
        buf[0, r, pl.ds(col * L, L)] = jnp.zeros((L,), _F32)
        return c

    lax.fori_loop(0, CH * 8, fill_zero, 0)
    for t in range(RPT // CH):
        pltpu.sync_copy(buf.at[0], acc_sh.at[pl.ds(sid * RPT + t * CH, CH)])
    plsc.subcore_barrier()

    def group(gi, c):
        _, gcount, win, roff = _group_geom(gi, t0, t1)
        pltpu.async_copy(src_hbm.at[pl.ds(win, GSZ)], sidx, sem).wait()
        pltpu.async_copy(dst_hbm.at[pl.ds(win, GSZ)], didx, sem).wait()

        pltpu.async_copy(g_hbm.at[sidx.at[roff]], buf.at[0], sem)

        def body(j, c2):
            b = j % 2
            pltpu.make_async_copy(g_hbm.at[sidx.at[roff + j]], buf.at[b],
                                  sem).wait()

            @pl.when(j + 1 < gcount)
            def _():
                pltpu.async_copy(g_hbm.at[sidx.at[roff + j + 1]],
                                 buf.at[(j + 1) % 2], sem)

            pltpu.sync_copy(buf.at[b], acc_sh.at[didx.at[roff + j]], add=True)
            return c2

        lax.fori_loop(0, gcount, body, 0)
        return c

    lax.fori_loop(0, _ngroups(t0, t1), group, 0)

    @pl.when(_is_tail_tile(cid, sid))
    def _():
        pltpu.async_copy(src_hbm.at[pl.ds(TAIL0, TAILC)],
                         sidx.at[pl.ds(0, TAILC)], sem).wait()
        pltpu.async_copy(dst_hbm.at[pl.ds(TAIL0, TAILC)],
                         didx.at[pl.ds(0, TAILC)], sem).wait()

        def tbody(j, c):
            pltpu.async_copy(g_hbm.at[sidx.at[j]], buf.at[0], sem).wait()
            pltpu.sync_copy(buf.at[0], acc_sh.at[didx.at[j]], add=True)
            return c

        lax.fori_loop(0, TAILC, tbody, 0)

    plsc.subcore_barrier()
    pltpu.sync_copy(acc_sh.at[pl.ds(sid * RPT, RPT)],
                    out_hbm.at[cid, pl.ds(sid * RPT, RPT)])


_agg_call = pl.kernel(
    _agg_body,
    out_type=jax.ShapeDtypeStruct((NC, NPAD, 128), _F32),
    mesh=_MESH,
    scratch_types=[
        pltpu.VMEM((GSZ, CH), jnp.int32),
        pltpu.VMEM((GSZ, CH), jnp.int32),
        pltpu.VMEM((2, CH, 128), _F32),
        pltpu.VMEM_SHARED((NPAD, 128), _F32),
        pltpu.SemaphoreType.DMA,
    ],
)


def _edge_body(a_hbm, b_hbm, src_hbm, dst_hbm, out_hbm, sidx, didx, buf,
               sem_a, sem_b, sem_o):
    cid = lax.axis_index("c")
    sid = lax.axis_index("s")
    t0, t1 = _chunk_range(cid, sid)

    def group(gi, c):
        gstart, gcount, win, roff = _group_geom(gi, t0, t1)
        pltpu.async_copy(src_hbm.at[pl.ds(win, GSZ)], sidx, sem_a).wait()
        pltpu.async_copy(dst_hbm.at[pl.ds(win, GSZ)], didx, sem_a).wait()

        pltpu.async_copy(a_hbm.at[sidx.at[roff]], buf.at[0], sem_a)

        def body(j, c2):
            b = j % 2
            pltpu.make_async_copy(a_hbm.at[sidx.at[roff + j]], buf.at[b],
                                  sem_a).wait()
            pltpu.async_copy(b_hbm.at[didx.at[roff + j]], buf.at[b], sem_b,
                             add=True)

            @pl.when(j >= 1)
            def _():
                pltpu.make_async_copy(
                    buf.at[1 - b],
                    out_hbm.at[pl.ds(pl.multiple_of((gstart + j - 1) * CH, CH),
                                     CH)],
                    sem_o).wait()

            @pl.when(j + 1 < gcount)
            def _():
                pltpu.async_copy(a_hbm.at[sidx.at[roff + j + 1]],
                                 buf.at[1 - b], sem_a)

            pltpu.make_async_copy(b_hbm.at[didx.at[roff + j]], buf.at[b],
                                  sem_b).wait()
            pltpu.async_copy(
                buf.at[b],
                out_hbm.at[pl.ds(pl.multiple_of((gstart + j) * CH, CH), CH)],
                sem_o)
            return c2

        lax.fori_loop(0, gcount, body, 0)
        pltpu.make_async_copy(
            buf.at[(gcount - 1) % 2],
            out_hbm.at[pl.ds(pl.multiple_of((gstart + gcount - 1) * CH, CH),
                             CH)],
            sem_o).wait()
        return c

    lax.fori_loop(0, _ngroups(t0, t1), group, 0)

    @pl.when(_is_tail_tile(cid, sid))
    def _():
        pltpu.async_copy(src_hbm.at[pl.ds(TAIL0, TAILC)],
                         sidx.at[pl.ds(0, TAILC)], sem_a).wait()
        pltpu.async_copy(dst_hbm.at[pl.ds(TAIL0, TAILC)],
                         didx.at[pl.ds(0, TAILC)], sem_a).wait()

        def tbody(j, c):
            pltpu.async_copy(a_hbm.at[sidx.at[j]], buf.at[0], sem_a).wait()
            pltpu.async_copy(b_hbm.at[didx.at[j]], buf.at[0], sem_b,
                             add=True).wait()
            pltpu.sync_copy(
                buf.at[0],
                out_hbm.at[pl.ds(pl.multiple_of((TAIL0 + j) * CH, CH), CH)])
            return c

        lax.fori_loop(0, TAILC, tbody, 0)


_edge_call = pl.kernel(
    _edge_body,
    out_type=jax.ShapeDtypeStruct((E, 128), _F32),
    mesh=_MESH,
    scratch_types=[
        pltpu.VMEM((GSZ, CH), jnp.int32),
        pltpu.VMEM((GSZ, CH), jnp.int32),
        pltpu.VMEM((2, CH, 128), _F32),
        pltpu.SemaphoreType.DMA,
        pltpu.SemaphoreType.DMA,
        pltpu.SemaphoreType.DMA,
    ],
)


_PREC = lax.Precision.HIGHEST


def _mm_body(x_ref, w_ref, o_ref):
    o_ref[:] = jnp.dot(x_ref[:], w_ref[:], preferred_element_type=_F32,
                       precision=_PREC)


def _tc_matmul(x, w, rows_per_block=2048):
    m = x.shape[0]
    grid = m // rows_per_block
    return pl.pallas_call(
        _mm_body,
        grid=(grid,),
        in_specs=[
            pl.BlockSpec((rows_per_block, x.shape[1]), lambda i: (i, 0)),
            pl.BlockSpec(w.shape, lambda i: (0, 0)),
        ],
        out_specs=pl.BlockSpec((rows_per_block, w.shape[1]), lambda i: (i, 0)),
        out_shape=jax.ShapeDtypeStruct((m, w.shape[1]), _F32),
    )(x, w)


def _scale_body(h_ref, d_ref, g_ref, dinv_ref):
    dinv = lax.rsqrt(d_ref[0] + d_ref[1] + 1.0)
    dinv_ref[:] = dinv
    g_ref[:] = h_ref[:] * dinv


def _tc_scale(h, deg):
    grid = NPAD // 2048
    return pl.pallas_call(
        _scale_body,
        grid=(grid,),
        in_specs=[
            pl.BlockSpec((2048, 128), lambda i: (i, 0)),
            pl.BlockSpec((2, 2048, 1), lambda i: (0, i, 0)),
        ],
        out_specs=[
            pl.BlockSpec((2048, 128), lambda i: (i, 0)),
            pl.BlockSpec((2048, 1), lambda i: (i, 0)),
        ],
        out_shape=[
            jax.ShapeDtypeStruct((NPAD, 128), _F32),
            jax.ShapeDtypeStruct((NPAD, 1), _F32),
        ],
    )(h, deg)


def _layer_body(a_ref, g_ref, dinv_ref, b_ref, w_ref, o_ref):
    dinv = dinv_ref[:]
    h = (a_ref[0] + a_ref[1] + g_ref[:]) * dinv + b_ref[:]
    h = jnp.maximum(h, 0.0)
    o_ref[:] = jnp.dot(h, w_ref[:], preferred_element_type=_F32,
                       precision=_PREC) * dinv


def _tc_layer(agg, g, dinv, b, w):
    grid = NPAD // 2048
    return pl.pallas_call(
        _layer_body,
        grid=(grid,),
        in_specs=[
            pl.BlockSpec((2, 2048, 128), lambda i: (0, i, 0)),
            pl.BlockSpec((2048, 128), lambda i: (i, 0)),
            pl.BlockSpec((2048, 1), lambda i: (i, 0)),
            pl.BlockSpec((1, 128), lambda i: (0, 0)),
            pl.BlockSpec((128, 128), lambda i: (0, 0)),
        ],
        out_specs=pl.BlockSpec((2048, 128), lambda i: (i, 0)),
        out_shape=jax.ShapeDtypeStruct((NPAD, 128), _F32),
    )(agg, g, dinv, b, w)


def _final_node_body(a_ref, g_ref, dinv_ref, b_ref, wa_ref, wb_ref,
                     oa_ref, ob_ref):
    dinv = dinv_ref[:]
    h = (a_ref[0] + a_ref[1] + g_ref[:]) * dinv + b_ref[:]
    h = jnp.maximum(h, 0.0)
    oa_ref[:] = jnp.dot(h, wa_ref[:], preferred_element_type=_F32,
                        precision=_PREC)
    ob_ref[:] = jnp.dot(h, wb_ref[:], preferred_element_type=_F32,
                        precision=_PREC)


def _tc_final_node(agg, g, dinv, b, wa, wb):
    grid = NPAD // 2048
    return pl.pallas_call(
        _final_node_body,
        grid=(grid,),
        in_specs=[
            pl.BlockSpec((2, 2048, 128), lambda i: (0, i, 0)),
            pl.BlockSpec((2048, 128), lambda i: (i, 0)),
            pl.BlockSpec((2048, 1), lambda i: (i, 0)),
            pl.BlockSpec((1, 128), lambda i: (0, 0)),
            pl.BlockSpec((128, 128), lambda i: (0, 0)),
            pl.BlockSpec((128, 128), lambda i: (0, 0)),
        ],
        out_specs=[
            pl.BlockSpec((2048, 128), lambda i: (i, 0)),
            pl.BlockSpec((2048, 128), lambda i: (i, 0)),
        ],
        out_shape=[
            jax.ShapeDtypeStruct((NPAD, 128), _F32),
            jax.ShapeDtypeStruct((NPAD, 128), _F32),
        ],
    )(agg, g, dinv, b, wa, wb)


def _edge_mlp_body(s_ref, b1_ref, w2_ref, b2_ref, o_ref):
    z = jnp.maximum(s_ref[:] + b1_ref[:], 0.0).astype(jnp.bfloat16)
    w2 = w2_ref[:].astype(jnp.bfloat16)
    o_ref[:] = lax.dot_general(w2, z, (((0,), (1,)), ((), ())),
                               preferred_element_type=_F32) + b2_ref[:]


def _tc_edge_mlp(s, bm1, wm2, bm2):
    rows = 12800
    grid = E // rows
    return pl.pallas_call(
        _edge_mlp_body,
        grid=(grid,),
        in_specs=[
            pl.BlockSpec((rows, 128), lambda i: (i, 0)),
            pl.BlockSpec((1, 128), lambda i: (0, 0)),
            pl.BlockSpec((128, 16), lambda i: (0, 0)),
            pl.BlockSpec((16, 1), lambda i: (0, 0)),
        ],
        out_specs=pl.BlockSpec((16, rows), lambda i: (0, i)),
        out_shape=jax.ShapeDtypeStruct((16, E), _F32),
    )(s, bm1, wm2, bm2)


def kernel(x, edge_index, W1, b1, W2, b2, Wm1, bm1, Wm2, bm2):
    xp = jnp.pad(x, ((0, NPAD - N), (0, 0)))
    srcc = edge_index[0].reshape(C_CHUNKS, CH)
    dstc = edge_index[1].reshape(C_CHUNKS, CH)

    h1 = _tc_matmul(xp, W1)
    deg = _deg_call(dstc)
    g1, dinv = _tc_scale(h1, deg.reshape(NC, NPAD, 1))
    agg1 = _agg_call(g1, srcc, dstc)
    g2 = _tc_layer(agg1, g1, dinv, b1.reshape(1, 128), W2)
    agg2 = _agg_call(g2, srcc, dstc)
    A, B = _tc_final_node(agg2, g2, dinv, b2.reshape(1, 128),
                          Wm1[:128], Wm1[128:])
    S = _edge_call(A, B, srcc, dstc)
    pred_t = _tc_edge_mlp(S, bm1.reshape(1, 128), Wm2, bm2.reshape(16, 1))
    return pred_t.T
